# Initial kernel scaffold; baseline (speedup 1.0000x reference)
import jax
import jax.numpy as jnp
from jax.experimental import pallas as pl


def kernel(x, pos, edge_index_local, edge_index_global, edge_attr_global, batch, W_atom, b_atom, W_bond, b_bond, W1, b1, W2, b2, Wv, W_lat, b_lat, Wn1, bn1, Wn2, bn2, Wg1, bg1, Wg2, bg2):
    raise NotImplementedError("write your pallas kernel here")


# speedup vs baseline: 8.9053x; 8.9053x over previous
import functools
import jax
import jax.numpy as jnp
from jax import lax
from jax.experimental import pallas as pl
from jax.experimental.pallas import tpu as pltpu
from jax.experimental.pallas import tpu_sc as plsc

N = 1024
FA = 16
FB = 5
EL = 16384
EG = 32768
NB = 32
SDIM = 256
VDIM = 64
EDIM = 32
RBF = 32
L = 5
LAT = 128
MH = 64
CUTOFF = 7.5
MOUT = SDIM + VDIM

ETOT = EL + EG
NW = 32
NC = 2
TBL_OFF = 512
NTBL = EG + TBL_OFF
GAMMA = (RBF / CUTOFF) ** 2

@functools.cache
def _get_mesh():
    return plsc.VectorSubcoreMesh(core_axis_name="c", subcore_axis_name="s")


def _wid():
    return lax.axis_index("s") * NC + lax.axis_index("c")


def _fill_idx(dst_buf, src_buf, src_base, nvec):
    def body(i, _):
        dst_buf[pl.ds(i * 16, 16)] = src_buf[pl.ds(src_base + i * 16, 16)]
        return 0
    lax.fori_loop(0, nvec, body, 0)


def _sc1a_body(eig, eil, posc, idmap, pos_src, pos_dst, cntl_part, cntg_part,
               idchunk, srcb, dstb, srcl, dstl, srcg, dstg, gbuf, ones_b, idx128,
               shared_l, shared_g):
    w = _wid()
    cid = lax.axis_index("c")
    sid = lax.axis_index("s")

    lo = w * (N * N // NW)
    zero16 = jnp.zeros((16,), jnp.int32)
    def zbody(i, _):
        idchunk[lax.shift_right_logical(i, 3), pl.ds((i & 7) * 16, 16)] = zero16
        return 0
    lax.fori_loop(0, (N * N // NW) // 16, zbody, 0)

    iota16 = lax.iota(jnp.int32, 16)
    CH = 2048
    def chunk_body(ci, _):
        pltpu.sync_copy(eig.at[0, pl.ds(ci * CH, CH)], srcb)
        pltpu.sync_copy(eig.at[1, pl.ds(ci * CH, CH)], dstb)
        def vec_body(j, _):
            s16 = srcb[pl.ds(j * 16, 16)]
            d16 = dstb[pl.ds(j * 16, 16)]
            k = s16 * N + d16 - lo
            ids = ci * CH + j * 16 + TBL_OFF + iota16
            m = (k >= 0) & (k < (N * N // NW))
            kc = jnp.where(m, k, 0)
            plsc.store_scatter(idchunk, [lax.shift_right_logical(kc, 7), kc & 127],
                               ids, mask=m)
            return 0
        lax.fori_loop(0, CH // 16, vec_body, 0)
        return 0
    lax.fori_loop(0, EG // CH, chunk_body, 0)
    pltpu.sync_copy(idchunk, idmap.at[pl.ds(w * (N * N // NW // 128), N * N // NW // 128)])

    lpt = EL // NW
    gpt = EG // NW
    pltpu.sync_copy(eil.at[0, pl.ds(w * lpt, lpt)], srcl)
    pltpu.sync_copy(eil.at[1, pl.ds(w * lpt, lpt)], dstl)
    pltpu.sync_copy(eig.at[0, pl.ds(w * gpt, gpt)], srcg)
    pltpu.sync_copy(eig.at[1, pl.ds(w * gpt, gpt)], dstg)

    for idxbuf, nloc, obase in ((srcl, lpt, w * lpt), (srcg, gpt, EL + w * gpt)):
        for j in range(nloc // 128):
            _fill_idx(idx128, idxbuf, j * 128, 8)
            pltpu.sync_copy(posc.at[idx128], gbuf)
            pltpu.sync_copy(gbuf, pos_src.at[pl.ds(obase + j * 128, 128)])
    for idxbuf, nloc, obase in ((dstl, lpt, w * lpt), (dstg, gpt, EL + w * gpt)):
        for j in range(nloc // 128):
            _fill_idx(idx128, idxbuf, j * 128, 8)
            pltpu.sync_copy(posc.at[idx128], gbuf)
            pltpu.sync_copy(gbuf, pos_dst.at[pl.ds(obase + j * 128, 128)])

    one16 = jnp.full((16,), 1.0, jnp.float32)
    zero16f = jnp.zeros((16,), jnp.float32)
    def obody(i, _):
        def obody2(jc, _2):
            ones_b[i, pl.ds(jc * 16, 16)] = one16
            gbuf[i, pl.ds(jc * 16, 16)] = zero16f
            return 0
        lax.fori_loop(0, 8, obody2, 0)
        return 0
    lax.fori_loop(0, 128, obody, 0)
    pltpu.sync_copy(gbuf.at[pl.ds(0, 64)], shared_l.at[pl.ds(sid * 64, 64)])
    pltpu.sync_copy(gbuf.at[pl.ds(0, 64)], shared_g.at[pl.ds(sid * 64, 64)])
    plsc.subcore_barrier()

    for idxbuf, nloc, shared in ((dstl, lpt, shared_l), (dstg, gpt, shared_g)):
        for j in range(nloc // 128):
            _fill_idx(idx128, idxbuf, j * 128, 8)
            pltpu.sync_copy(ones_b, shared.at[idx128], add=True)
    plsc.subcore_barrier()

    pltpu.sync_copy(shared_l.at[pl.ds(sid * 64, 64)], cntl_part.at[cid, pl.ds(sid * 64, 64)])
    pltpu.sync_copy(shared_g.at[pl.ds(sid * 64, 64)], cntg_part.at[cid, pl.ds(sid * 64, 64)])


def _sc1a(eig, eil, posc):
    f = functools.partial(
        pl.kernel,
        out_type=(
            jax.ShapeDtypeStruct((N * N // 128, 128), jnp.int32),
            jax.ShapeDtypeStruct((ETOT, 128), jnp.float32),
            jax.ShapeDtypeStruct((ETOT, 128), jnp.float32),
            jax.ShapeDtypeStruct((NC, N, 128), jnp.float32),
            jax.ShapeDtypeStruct((NC, N, 128), jnp.float32),
        ),
        mesh=_get_mesh(),
        compiler_params=pltpu.CompilerParams(needs_layout_passes=False, use_tc_tiling_on_sc=False),
        scratch_types=[
            pltpu.VMEM((N * N // NW // 128, 128), jnp.int32),
            pltpu.VMEM((2048,), jnp.int32),
            pltpu.VMEM((2048,), jnp.int32),
            pltpu.VMEM((EL // NW,), jnp.int32),
            pltpu.VMEM((EL // NW,), jnp.int32),
            pltpu.VMEM((EG // NW,), jnp.int32),
            pltpu.VMEM((EG // NW,), jnp.int32),
            pltpu.VMEM((128, 128), jnp.float32),
            pltpu.VMEM((128, 128), jnp.float32),
            pltpu.VMEM((128,), jnp.int32),
            pltpu.VMEM_SHARED((N, 128), jnp.float32),
            pltpu.VMEM_SHARED((N, 128), jnp.float32),
        ],
    )
    return f(_sc1a_body)(eig, eil, posc)


def _sc1b_body(idmap, eil, e_table, e_l, srcl, dstl, keyrow, keycol, idx128,
               lidb, rowsbuf, erows):
    w = _wid()
    lpt = EL // NW
    iota16 = lax.iota(jnp.int32, 16)
    pltpu.sync_copy(eil.at[0, pl.ds(w * lpt, lpt)], srcl)
    pltpu.sync_copy(eil.at[1, pl.ds(w * lpt, lpt)], dstl)
    def kbody(i, _):
        key = srcl[pl.ds(i * 16, 16)] * N + dstl[pl.ds(i * 16, 16)]
        keyrow[pl.ds(i * 16, 16)] = lax.shift_right_logical(key, 7)
        keycol[pl.ds(i * 16, 16)] = key & 127
        return 0
    lax.fori_loop(0, lpt // 16, kbody, 0)
    for j in range(lpt // 128):
        _fill_idx(idx128, keyrow, j * 128, 8)
        pltpu.sync_copy(idmap.at[idx128], rowsbuf)
        for t in range(8):
            r16 = t * 16 + iota16
            c16 = keycol[pl.ds(j * 128 + t * 16, 16)]
            lidb[pl.ds(t * 16, 16)] = plsc.load_gather(rowsbuf, [r16, c16])
        pltpu.sync_copy(e_table.at[lidb], erows)
        pltpu.sync_copy(erows, e_l.at[pl.ds(w * lpt + j * 128, 128)])


def _sc1b(idmap, eil, e_table):
    f = functools.partial(
        pl.kernel,
        out_type=jax.ShapeDtypeStruct((EL, 128), jnp.float32),
        mesh=_get_mesh(),
        compiler_params=pltpu.CompilerParams(needs_layout_passes=False, use_tc_tiling_on_sc=False),
        scratch_types=[
            pltpu.VMEM((EL // NW,), jnp.int32),
            pltpu.VMEM((EL // NW,), jnp.int32),
            pltpu.VMEM((EL // NW,), jnp.int32),
            pltpu.VMEM((EL // NW,), jnp.int32),
            pltpu.VMEM((128,), jnp.int32),
            pltpu.VMEM((128,), jnp.int32),
            pltpu.VMEM((128, 128), jnp.int32),
            pltpu.VMEM((128, 128), jnp.float32),
        ],
    )
    return f(_sc1b_body)(idmap, eil, e_table)


def _make_sc2(E):
    ept = E // NW

    def body(ei, PQ, PS, QD, srcb, dstb, idx128, bufS, bufD):
        w = _wid()
        pltpu.sync_copy(ei.at[0, pl.ds(w * ept, ept)], srcb)
        pltpu.sync_copy(ei.at[1, pl.ds(w * ept, ept)], dstb)
        for j in range(ept // 128):
            _fill_idx(idx128, srcb, j * 128, 8)
            pltpu.sync_copy(PQ.at[idx128], bufS)
            pltpu.sync_copy(bufS, PS.at[pl.ds(w * ept + j * 128, 128)])
            _fill_idx(idx128, dstb, j * 128, 8)
            pltpu.sync_copy(PQ.at[idx128], bufD)
            pltpu.sync_copy(bufD, QD.at[pl.ds(w * ept + j * 128, 128)])

    f = functools.partial(
        pl.kernel,
        out_type=(
            jax.ShapeDtypeStruct((E, 128), jnp.float32),
            jax.ShapeDtypeStruct((E, 128), jnp.float32),
        ),
        mesh=_get_mesh(),
        compiler_params=pltpu.CompilerParams(needs_layout_passes=False, use_tc_tiling_on_sc=False),
        scratch_types=[
            pltpu.VMEM((ept,), jnp.int32),
            pltpu.VMEM((ept,), jnp.int32),
            pltpu.VMEM((128,), jnp.int32),
            pltpu.VMEM((128, 128), jnp.float32),
            pltpu.VMEM((128, 128), jnp.float32),
        ],
    )
    return f(body)


MSGW = SDIM + 3 * VDIM
MSGP = 512


def _make_sc3(E):
    ept = E // NW

    def body(msg, ei, agg, dstb, idx64, bufM, shared):
        w = _wid()
        cid = lax.axis_index("c")
        sid = lax.axis_index("s")
        zero16f = jnp.zeros((16,), jnp.float32)
        def zb(i, _):
            def zb2(jc, _2):
                bufM[i, pl.ds(jc * 16, 16)] = zero16f
                return 0
            lax.fori_loop(0, MSGP // 16, zb2, 0)
            return 0
        lax.fori_loop(0, 64, zb, 0)
        pltpu.sync_copy(bufM, shared.at[pl.ds(sid * 64, 64)])
        plsc.subcore_barrier()

        pltpu.sync_copy(ei.at[1, pl.ds(w * ept, ept)], dstb)
        for k in range(ept // 64):
            pltpu.sync_copy(msg.at[pl.ds(w * ept + k * 64, 64)], bufM)
            _fill_idx(idx64, dstb, k * 64, 4)
            pltpu.sync_copy(bufM, shared.at[idx64], add=True)
        plsc.subcore_barrier()
        pltpu.sync_copy(shared.at[pl.ds(sid * 64, 64)], agg.at[cid, pl.ds(sid * 64, 64)])

    f = functools.partial(
        pl.kernel,
        out_type=jax.ShapeDtypeStruct((NC, N, MSGP), jnp.float32),
        mesh=_get_mesh(),
        compiler_params=pltpu.CompilerParams(needs_layout_passes=False, use_tc_tiling_on_sc=False),
        scratch_types=[
            pltpu.VMEM((ept,), jnp.int32),
            pltpu.VMEM((64,), jnp.int32),
            pltpu.VMEM((64, MSGP), jnp.float32),
            pltpu.VMEM_SHARED((N, MSGP), jnp.float32),
        ],
    )
    return f(body)


def _silu(x):
    return x * (1.0 / (1.0 + jnp.exp(-x)))


def _dot(a, b):
    return jax.lax.dot_general(a, b, (((1,), (0,)), ((), ())),
                               preferred_element_type=jnp.float32)


def _dotT(a, b):
    return jax.lax.dot_general(a, b, (((0,), (0,)), ((), ())),
                               preferred_element_type=jnp.float32)


def _tc_prep_a_body(x, W_atom, b_atom, posp, batch_row, W1s0, W1d0,
                    s0_o, PQ0_o, posc_o):
    s0 = _dot(x[...], W_atom[...]) + b_atom[...]
    s0_o[...] = s0
    PQ0_o[...] = jnp.concatenate([_dot(s0, W1s0[...]), _dot(s0, W1d0[...])], axis=1)
    M = (batch_row[...] == lax.broadcasted_iota(jnp.int32, (NB, N), 0)).astype(jnp.float32)
    cnt_b = jnp.sum(M, axis=1, keepdims=True)
    pos_mean = _dot(M, posp[...]) / jnp.maximum(cnt_b, 1.0)
    posc = posp[...] - _dotT(M, pos_mean)
    posc_o[...] = jnp.concatenate(
        [posc, jnp.zeros((N, 112), jnp.float32)], axis=1)


def _tc_prep_a(x, W_atom, b_atom, posp, batch_row, W1s0, W1d0):
    return pl.pallas_call(
        _tc_prep_a_body,
        out_shape=(
            jax.ShapeDtypeStruct((N, SDIM), jnp.float32),
            jax.ShapeDtypeStruct((N, 128), jnp.float32),
            jax.ShapeDtypeStruct((N, 128), jnp.float32),
        ),
    )(x, W_atom, b_atom, posp, batch_row, W1s0, W1d0)


def _tc_prep_b_body(ea, W_bond, b_bond, out):
    i = pl.program_id(0)
    et = _dot(ea[...], W_bond[...]) + b_bond[...]
    et = jnp.where(i == 0, jnp.zeros_like(et), et)
    out[...] = jnp.concatenate([et, jnp.zeros((et.shape[0], 128 - EDIM), jnp.float32)], axis=1)


def _tc_prep_b(ea8, W_bond8, b_bond):
    nb = NTBL // 512
    return pl.pallas_call(
        _tc_prep_b_body,
        grid=(nb,),
        in_specs=[
            pl.BlockSpec((512, 8), lambda i: (jnp.maximum(i - 1, 0), 0)),
            pl.BlockSpec((8, EDIM), lambda i: (0, 0)),
            pl.BlockSpec((1, EDIM), lambda i: (0, 0)),
        ],
        out_specs=pl.BlockSpec((512, 128), lambda i: (i, 0)),
        out_shape=jax.ShapeDtypeStruct((NTBL, 128), jnp.float32),
    )(ea8, W_bond8, b_bond)


def _tc_geom_body(ps, pd, e, G_o, rn_o):
    psv = ps[...]
    pdv = pd[...]
    r = pdv - psv
    d2 = jnp.sum(r * r, axis=1, keepdims=True)
    a = jnp.sum(psv * pdv, axis=1, keepdims=True)
    d = jnp.sqrt(jnp.maximum(d2, 1e-6))
    rn = r / d
    rn_o[...] = rn[:, :16]
    mus = (CUTOFF / (RBF - 1)) * lax.broadcasted_iota(jnp.int32, (1, RBF), 1).astype(jnp.float32)
    rb = jnp.exp(-GAMMA * (d - mus) ** 2)
    G_o[...] = jnp.concatenate(
        [rb, e[:, :EDIM], a, jnp.zeros((rb.shape[0], 128 - RBF - EDIM - 1), jnp.float32)], axis=1)


def _tc_geom(pos_src, pos_dst, e_cat):
    nb = ETOT // 512
    return pl.pallas_call(
        _tc_geom_body,
        grid=(nb,),
        in_specs=[
            pl.BlockSpec((512, 128), lambda i: (i, 0)),
            pl.BlockSpec((512, 128), lambda i: (i, 0)),
            pl.BlockSpec((512, 128), lambda i: (i, 0)),
        ],
        out_specs=(
            pl.BlockSpec((512, 128), lambda i: (i, 0)),
            pl.BlockSpec((512, 16), lambda i: (i, 0)),
        ),
        out_shape=(
            jax.ShapeDtypeStruct((ETOT, 128), jnp.float32),
            jax.ShapeDtypeStruct((ETOT, 16), jnp.float32),
        ),
    )(pos_src, pos_dst, e_cat)


def _tc_et_body(G, W1g, b1, out):
    g = G[...]
    for l in range(L):
        out[l, :, :] = _dot(g, W1g[l]) + b1[l][None, :]


def _tc_et(G, W1g_pad, b1):
    nb = ETOT // 512
    return pl.pallas_call(
        _tc_et_body,
        grid=(nb,),
        in_specs=[
            pl.BlockSpec((512, 128), lambda i: (i, 0)),
            pl.BlockSpec((L, 128, MH), lambda i: (0, 0, 0)),
            pl.BlockSpec((L, MH), lambda i: (0, 0)),
        ],
        out_specs=pl.BlockSpec((L, 512, MH), lambda i: (0, i, 0)),
        out_shape=jax.ShapeDtypeStruct((L, ETOT, MH), jnp.float32),
    )(G, W1g_pad, b1)


def _make_tc_b(E, base, l):
    nb = E // 512

    def body(ET, PS, QD, rn, W2l, b2l, out):
        h1 = _silu(ET[0] + PS[:, :MH] + QD[:, MH:])
        h = _dot(h1, W2l[...]) + b2l[...]
        ms = h[:, :SDIM]
        mv = h[:, SDIM:]
        rnv = rn[...]
        out[...] = jnp.concatenate(
            [ms, mv * rnv[:, 0:1], mv * rnv[:, 1:2], mv * rnv[:, 2:3],
             jnp.zeros((h.shape[0], MSGP - MSGW), jnp.float32)], axis=1)

    return pl.pallas_call(
        body,
        grid=(nb,),
        in_specs=[
            pl.BlockSpec((1, 512, MH), lambda i: (l, base // 512 + i, 0)),
            pl.BlockSpec((512, 128), lambda i: (i, 0)),
            pl.BlockSpec((512, 128), lambda i: (i, 0)),
            pl.BlockSpec((512, 16), lambda i: (base // 512 + i, 0)),
            pl.BlockSpec((MH, MOUT), lambda i: (0, 0)),
            pl.BlockSpec((1, MOUT), lambda i: (0, 0)),
        ],
        out_specs=pl.BlockSpec((512, MSGP), lambda i: (i, 0)),
        out_shape=jax.ShapeDtypeStruct((E, MSGP), jnp.float32),
    )


def _tc_d1_body(s, v, aggL, cntl, W1s, W1d, s_mid_o, v_mid_o, PQ_o):
    s_mid = s[...] + aggL[0, :, :SDIM] + aggL[1, :, :SDIM]
    cnt = jnp.maximum(cntl[0, :, 0:1] + cntl[1, :, 0:1], 1.0)
    v_mid_o[...] = v[...] + (aggL[0, :, SDIM:MSGW] + aggL[1, :, SDIM:MSGW]) / cnt
    s_mid_o[...] = s_mid
    PQ_o[...] = jnp.concatenate([_dot(s_mid, W1s[...]), _dot(s_mid, W1d[...])], axis=1)


def _tc_d1(s, v, aggL, cntl, W1s, W1d):
    return pl.pallas_call(
        _tc_d1_body,
        out_shape=(
            jax.ShapeDtypeStruct((N, SDIM), jnp.float32),
            jax.ShapeDtypeStruct((N, 3 * VDIM), jnp.float32),
            jax.ShapeDtypeStruct((N, 128), jnp.float32),
        ),
    )(s, v, aggL, cntl, W1s, W1d)


def _tc_d2_body(s_mid, v_mid, aggG, cntg, Wvl, W1s, W1d,
                s_o, v_o, PQ_o):
    s2 = s_mid[...] + aggG[0, :, :SDIM] + aggG[1, :, :SDIM]
    cnt = jnp.maximum(cntg[0, :, 0:1] + cntg[1, :, 0:1], 1.0)
    v_new = v_mid[...] + (aggG[0, :, SDIM:MSGW] + aggG[1, :, SDIM:MSGW]) / cnt
    v_o[...] = v_new
    vn = jnp.sqrt(v_new[:, :VDIM] ** 2 + v_new[:, VDIM:2 * VDIM] ** 2
                  + v_new[:, 2 * VDIM:] ** 2 + 1e-6)
    sp = s2 + _dot(vn, Wvl[...])
    m = jnp.mean(sp, axis=1, keepdims=True)
    c = sp - m
    var = jnp.mean(c * c, axis=1, keepdims=True)
    s_new = c / jnp.sqrt(var + 1e-5)
    s_o[...] = s_new
    PQ_o[...] = jnp.concatenate([_dot(s_new, W1s[...]), _dot(s_new, W1d[...])], axis=1)


def _tc_d2(s_mid, v_mid, aggG, cntg, Wvl, W1s, W1d):
    return pl.pallas_call(
        _tc_d2_body,
        out_shape=(
            jax.ShapeDtypeStruct((N, SDIM), jnp.float32),
            jax.ShapeDtypeStruct((N, 3 * VDIM), jnp.float32),
            jax.ShapeDtypeStruct((N, 128), jnp.float32),
        ),
    )(s_mid, v_mid, aggG, cntg, Wvl, W1s, W1d)


def _tc_readout_body(s, batch_col, W_lat, b_lat, Wn1, bn1, Wn2, bn2,
                     Wg1, bg1, Wg2, bg2, pooled_o):
    out = _dot(s[...], W_lat[...]) + b_lat[...]
    g1 = _silu(_dot(out, Wg1[...]) + bg1[...])
    gate = _dot(g1, Wg2[...]) + bg2[...]
    nd = _silu(_dot(out, Wn1[...]) + bn1[...])
    nd = _dot(nd, Wn2[...]) + bn2[...]
    MT = (batch_col[:, 0:1] == lax.broadcasted_iota(jnp.int32, (N, NB), 1))
    MTf = MT.astype(jnp.float32)
    masked = jnp.where(MT, jnp.broadcast_to(gate, (N, NB)), -1e30)
    gmax = jnp.max(masked, axis=0, keepdims=True)
    gmax_pn = jax.lax.dot_general(MTf, gmax, (((1,), (1,)), ((), ())),
                                  preferred_element_type=jnp.float32)
    ge = jnp.exp(gate - gmax_pn)
    gden = _dotT(MTf, ge)
    gden_pn = _dot(MTf, gden)
    gate_n = ge / jnp.maximum(gden_pn, 1e-16)
    pooled_o[...] = _dotT(MTf, gate_n * nd)


def _tc_readout(s, batch_col, W_lat, b_lat, Wn1, bn1, Wn2, bn2, Wg1, bg1, Wg2, bg2):
    return pl.pallas_call(
        _tc_readout_body,
        out_shape=jax.ShapeDtypeStruct((NB, LAT), jnp.float32),
    )(s, batch_col, W_lat, b_lat, Wn1, bn1, Wn2, bn2, Wg1, bg1, Wg2, bg2)


@jax.jit
def kernel(x, pos, edge_index_local, edge_index_global, edge_attr_global, batch,
           W_atom, b_atom, W_bond, b_bond, W1, b1, W2, b2, Wv, W_lat, b_lat,
           Wn1, bn1, Wn2, bn2, Wg1, bg1, Wg2, bg2):
    eil = edge_index_local.astype(jnp.int32)
    eig = edge_index_global.astype(jnp.int32)
    batch_i = batch.astype(jnp.int32)
    posp = jnp.pad(pos, ((0, 0), (0, 16 - 3)))
    batch_row = batch_i.reshape(1, N)
    batch_col = jnp.broadcast_to(batch_i.reshape(N, 1), (N, 8))
    ea8 = jnp.pad(edge_attr_global, ((0, 0), (0, 8 - FB)))
    W_bond8 = jnp.pad(W_bond, ((0, 8 - FB), (0, 0)))

    W1s = W1[:, :SDIM, :]
    W1d = W1[:, SDIM:2 * SDIM, :]
    W1g_pad = jnp.zeros((L, 128, MH), jnp.float32)
    W1g_pad = W1g_pad.at[:, :RBF, :].set(W1[:, 2 * SDIM:2 * SDIM + RBF, :])
    W1g_pad = W1g_pad.at[:, RBF:RBF + EDIM, :].set(W1[:, 2 * SDIM + RBF:2 * SDIM + RBF + EDIM, :])
    W1g_pad = W1g_pad.at[:, RBF + EDIM, :].set(W1[:, 2 * SDIM + RBF + EDIM, :])

    s0, PQ, posc = _tc_prep_a(x, W_atom, b_atom.reshape(1, SDIM), posp,
                                batch_row, W1s[0], W1d[0])
    e_table = _tc_prep_b(ea8, W_bond8, b_bond.reshape(1, EDIM))

    idmap, pos_src, pos_dst, cntl_part, cntg_part = _sc1a(eig, eil, posc)
    e_l = _sc1b(idmap, eil, e_table)

    e_cat = jnp.concatenate([e_l, e_table[TBL_OFF:]], axis=0)
    G, rn16 = _tc_geom(pos_src, pos_dst, e_cat)
    ET_all = _tc_et(G, W1g_pad, b1)

    sc2_l = _make_sc2(EL)
    sc2_g = _make_sc2(EG)
    sc3_l = _make_sc3(EL)
    sc3_g = _make_sc3(EG)

    s = s0
    v = jnp.zeros((N, 3 * VDIM), jnp.float32)
    for l in range(L):
        PS, QD = sc2_l(eil, PQ)
        msg = _make_tc_b(EL, 0, l)(ET_all, PS, QD, rn16, W2[l], b2[l].reshape(1, MOUT))
        aggL = sc3_l(msg, eil)
        s, v, PQ = _tc_d1(s, v, aggL, cntl_part, W1s[l], W1d[l])
        PS, QD = sc2_g(eig, PQ)
        msg = _make_tc_b(EG, EL, l)(ET_all, PS, QD, rn16, W2[l], b2[l].reshape(1, MOUT))
        aggG = sc3_g(msg, eig)
        ln = min(l + 1, L - 1)
        s, v, PQ = _tc_d2(s, v, aggG, cntg_part, Wv[l], W1s[ln], W1d[ln])

    return _tc_readout(s, batch_col, W_lat, b_lat.reshape(1, LAT),
                       Wn1, bn1.reshape(1, LAT), Wn2, bn2.reshape(1, LAT),
                       Wg1, bg1.reshape(1, LAT), Wg2, bg2.reshape(1, 1))


# speedup vs baseline: 10.7687x; 1.2092x over previous
import functools
import jax
import jax.numpy as jnp
from jax import lax
from jax.experimental import pallas as pl
from jax.experimental.pallas import tpu as pltpu
from jax.experimental.pallas import tpu_sc as plsc

N = 1024
FA = 16
FB = 5
EL = 16384
EG = 32768
NB = 32
SDIM = 256
VDIM = 64
EDIM = 32
RBF = 32
L = 5
LAT = 128
MH = 64
CUTOFF = 7.5
MOUT = SDIM + VDIM

ETOT = EL + EG
NW = 32
NC = 2
TBL_OFF = 512
NTBL = EG + TBL_OFF
GAMMA = (RBF / CUTOFF) ** 2

@functools.cache
def _get_mesh():
    return plsc.VectorSubcoreMesh(core_axis_name="c", subcore_axis_name="s")


def _wid():
    return lax.axis_index("s") * NC + lax.axis_index("c")


def _fill_idx(dst_buf, src_buf, src_base, nvec):
    def body(i, _):
        dst_buf[pl.ds(i * 16, 16)] = src_buf[pl.ds(src_base + i * 16, 16)]
        return 0
    lax.fori_loop(0, nvec, body, 0)


def _sc1a_body(eig, eil, posc, idmap, pos_src, pos_dst, cntl_part, cntg_part,
               idchunk, srcb, dstb, srcl, dstl, srcg, dstg, gbuf, ones_b, idx128,
               shared_l, shared_g):
    w = _wid()
    cid = lax.axis_index("c")
    sid = lax.axis_index("s")

    lo = w * (N * N // NW)
    zero16 = jnp.zeros((16,), jnp.int32)
    def zbody(i, _):
        idchunk[lax.shift_right_logical(i, 3), pl.ds((i & 7) * 16, 16)] = zero16
        return 0
    lax.fori_loop(0, (N * N // NW) // 16, zbody, 0)

    iota16 = lax.iota(jnp.int32, 16)
    CH = 2048
    def chunk_body(ci, _):
        pltpu.sync_copy(eig.at[0, pl.ds(ci * CH, CH)], srcb)
        pltpu.sync_copy(eig.at[1, pl.ds(ci * CH, CH)], dstb)
        def vec_body(j, _):
            s16 = srcb[pl.ds(j * 16, 16)]
            d16 = dstb[pl.ds(j * 16, 16)]
            k = s16 * N + d16 - lo
            ids = ci * CH + j * 16 + TBL_OFF + iota16
            m = (k >= 0) & (k < (N * N // NW))
            kc = jnp.where(m, k, 0)
            plsc.store_scatter(idchunk, [lax.shift_right_logical(kc, 7), kc & 127],
                               ids, mask=m)
            return 0
        lax.fori_loop(0, CH // 16, vec_body, 0)
        return 0
    lax.fori_loop(0, EG // CH, chunk_body, 0)
    pltpu.sync_copy(idchunk, idmap.at[pl.ds(w * (N * N // NW // 128), N * N // NW // 128)])

    lpt = EL // NW
    gpt = EG // NW
    pltpu.sync_copy(eil.at[0, pl.ds(w * lpt, lpt)], srcl)
    pltpu.sync_copy(eil.at[1, pl.ds(w * lpt, lpt)], dstl)
    pltpu.sync_copy(eig.at[0, pl.ds(w * gpt, gpt)], srcg)
    pltpu.sync_copy(eig.at[1, pl.ds(w * gpt, gpt)], dstg)

    for idxbuf, nloc, obase in ((srcl, lpt, w * lpt), (srcg, gpt, EL + w * gpt)):
        for j in range(nloc // 128):
            _fill_idx(idx128, idxbuf, j * 128, 8)
            pltpu.sync_copy(posc.at[idx128], gbuf)
            pltpu.sync_copy(gbuf, pos_src.at[pl.ds(obase + j * 128, 128)])
    for idxbuf, nloc, obase in ((dstl, lpt, w * lpt), (dstg, gpt, EL + w * gpt)):
        for j in range(nloc // 128):
            _fill_idx(idx128, idxbuf, j * 128, 8)
            pltpu.sync_copy(posc.at[idx128], gbuf)
            pltpu.sync_copy(gbuf, pos_dst.at[pl.ds(obase + j * 128, 128)])

    one16 = jnp.full((16,), 1.0, jnp.float32)
    zero16f = jnp.zeros((16,), jnp.float32)
    def obody(i, _):
        def obody2(jc, _2):
            ones_b[i, pl.ds(jc * 16, 16)] = one16
            gbuf[i, pl.ds(jc * 16, 16)] = zero16f
            return 0
        lax.fori_loop(0, 8, obody2, 0)
        return 0
    lax.fori_loop(0, 128, obody, 0)
    pltpu.sync_copy(gbuf.at[pl.ds(0, 64)], shared_l.at[pl.ds(sid * 64, 64)])
    pltpu.sync_copy(gbuf.at[pl.ds(0, 64)], shared_g.at[pl.ds(sid * 64, 64)])
    plsc.subcore_barrier()

    for idxbuf, nloc, shared in ((dstl, lpt, shared_l), (dstg, gpt, shared_g)):
        for j in range(nloc // 128):
            _fill_idx(idx128, idxbuf, j * 128, 8)
            pltpu.sync_copy(ones_b, shared.at[idx128], add=True)
    plsc.subcore_barrier()

    pltpu.sync_copy(shared_l.at[pl.ds(sid * 64, 64)], cntl_part.at[cid, pl.ds(sid * 64, 64)])
    pltpu.sync_copy(shared_g.at[pl.ds(sid * 64, 64)], cntg_part.at[cid, pl.ds(sid * 64, 64)])


def _sc1a(eig, eil, posc):
    f = functools.partial(
        pl.kernel,
        out_type=(
            jax.ShapeDtypeStruct((N * N // 128, 128), jnp.int32),
            jax.ShapeDtypeStruct((ETOT, 128), jnp.float32),
            jax.ShapeDtypeStruct((ETOT, 128), jnp.float32),
            jax.ShapeDtypeStruct((NC, N, 128), jnp.float32),
            jax.ShapeDtypeStruct((NC, N, 128), jnp.float32),
        ),
        mesh=_get_mesh(),
        compiler_params=pltpu.CompilerParams(needs_layout_passes=False, use_tc_tiling_on_sc=False),
        scratch_types=[
            pltpu.VMEM((N * N // NW // 128, 128), jnp.int32),
            pltpu.VMEM((2048,), jnp.int32),
            pltpu.VMEM((2048,), jnp.int32),
            pltpu.VMEM((EL // NW,), jnp.int32),
            pltpu.VMEM((EL // NW,), jnp.int32),
            pltpu.VMEM((EG // NW,), jnp.int32),
            pltpu.VMEM((EG // NW,), jnp.int32),
            pltpu.VMEM((128, 128), jnp.float32),
            pltpu.VMEM((128, 128), jnp.float32),
            pltpu.VMEM((128,), jnp.int32),
            pltpu.VMEM_SHARED((N, 128), jnp.float32),
            pltpu.VMEM_SHARED((N, 128), jnp.float32),
        ],
    )
    return f(_sc1a_body)(eig, eil, posc)


def _sc1b_body(idmap, eil, e_table, e_l, srcl, dstl, keyrow, keycol, idx128,
               lidb, rowsbuf, erows):
    w = _wid()
    lpt = EL // NW
    iota16 = lax.iota(jnp.int32, 16)
    pltpu.sync_copy(eil.at[0, pl.ds(w * lpt, lpt)], srcl)
    pltpu.sync_copy(eil.at[1, pl.ds(w * lpt, lpt)], dstl)
    def kbody(i, _):
        key = srcl[pl.ds(i * 16, 16)] * N + dstl[pl.ds(i * 16, 16)]
        keyrow[pl.ds(i * 16, 16)] = lax.shift_right_logical(key, 7)
        keycol[pl.ds(i * 16, 16)] = key & 127
        return 0
    lax.fori_loop(0, lpt // 16, kbody, 0)
    for j in range(lpt // 128):
        _fill_idx(idx128, keyrow, j * 128, 8)
        pltpu.sync_copy(idmap.at[idx128], rowsbuf)
        for t in range(8):
            r16 = t * 16 + iota16
            c16 = keycol[pl.ds(j * 128 + t * 16, 16)]
            lidb[pl.ds(t * 16, 16)] = plsc.load_gather(rowsbuf, [r16, c16])
        pltpu.sync_copy(e_table.at[lidb], erows)
        pltpu.sync_copy(erows, e_l.at[pl.ds(w * lpt + j * 128, 128)])


def _sc1b(idmap, eil, e_table):
    f = functools.partial(
        pl.kernel,
        out_type=jax.ShapeDtypeStruct((EL, 128), jnp.float32),
        mesh=_get_mesh(),
        compiler_params=pltpu.CompilerParams(needs_layout_passes=False, use_tc_tiling_on_sc=False),
        scratch_types=[
            pltpu.VMEM((EL // NW,), jnp.int32),
            pltpu.VMEM((EL // NW,), jnp.int32),
            pltpu.VMEM((EL // NW,), jnp.int32),
            pltpu.VMEM((EL // NW,), jnp.int32),
            pltpu.VMEM((128,), jnp.int32),
            pltpu.VMEM((128,), jnp.int32),
            pltpu.VMEM((128, 128), jnp.int32),
            pltpu.VMEM((128, 128), jnp.float32),
        ],
    )
    return f(_sc1b_body)(idmap, eil, e_table)


def _make_sc2(E):
    ept = E // NW
    nj = ept // 128
    ntask = 2 * nj

    def body(ei, PQ, PS, QD, srcb, dstb, idx2,
             buf0, buf1, buf2, buf3, g0, g1, g2, g3, w0, w1, w2, w3):
        w = _wid()
        bufs = [buf0, buf1, buf2, buf3]
        gsems = [g0, g1, g2, g3]
        wsems = [w0, w1, w2, w3]
        pltpu.sync_copy(ei.at[0, pl.ds(w * ept, ept)], srcb)
        pltpu.sync_copy(ei.at[1, pl.ds(w * ept, ept)], dstb)
        for j in range(nj):
            def fb(i, _, j=j):
                idx2[j, pl.ds(i * 16, 16)] = srcb[pl.ds(j * 128 + i * 16, 16)]
                idx2[nj + j, pl.ds(i * 16, 16)] = dstb[pl.ds(j * 128 + i * 16, 16)]
                return 0
            lax.fori_loop(0, 8, fb, 0)

        def out_at(t):
            if t < nj:
                return PS.at[pl.ds(w * ept + t * 128, 128)]
            return QD.at[pl.ds(w * ept + (t - nj) * 128, 128)]

        gd = [None] * 4
        wd = [None] * 4
        for t in range(min(4, ntask)):
            b = t % 4
            gd[b] = pltpu.async_copy(PQ.at[idx2.at[t]], bufs[b], gsems[b])
        for t in range(ntask):
            b = t % 4
            gd[b].wait()
            wd[b] = pltpu.async_copy(bufs[b], out_at(t), wsems[b])
            nt = t + 4
            if nt < ntask:
                wd[b].wait()
                wd[b] = None
                gd[b] = pltpu.async_copy(PQ.at[idx2.at[nt]], bufs[b], gsems[b])
        for b in range(4):
            if wd[b] is not None:
                wd[b].wait()

    f = functools.partial(
        pl.kernel,
        out_type=(
            jax.ShapeDtypeStruct((E, 128), jnp.float32),
            jax.ShapeDtypeStruct((E, 128), jnp.float32),
        ),
        mesh=_get_mesh(),
        compiler_params=pltpu.CompilerParams(needs_layout_passes=False, use_tc_tiling_on_sc=False),
        scratch_types=[
            pltpu.VMEM((ept,), jnp.int32),
            pltpu.VMEM((ept,), jnp.int32),
            pltpu.VMEM((ntask, 128), jnp.int32),
        ] + [pltpu.VMEM((128, 128), jnp.float32)] * 4
          + [pltpu.SemaphoreType.DMA] * 8,
    )
    return f(body)


MSGW = SDIM + 3 * VDIM


def _make_sc3(E, W, base):
    ept = E // NW
    ntask = ept // 64

    def body(msg, ei, zrows, agg, dstb,
             idx0, idx1, idx2_, idx3, buf0, buf1, buf2, buf3,
             g0, g1, g2, g3, s0, s1, s2, s3, shared):
        w = _wid()
        cid = lax.axis_index("c")
        sid = lax.axis_index("s")
        bufs = [buf0, buf1, buf2, buf3]
        idxs = [idx0, idx1, idx2_, idx3]
        gsems = [g0, g1, g2, g3]
        ssems = [s0, s1, s2, s3]
        pltpu.sync_copy(zrows, shared.at[pl.ds(sid * 64, 64)])
        plsc.subcore_barrier()
        pltpu.sync_copy(ei.at[1, pl.ds(w * ept, ept)], dstb)

        gd = [None] * 4
        sd = [None] * 4
        for t in range(min(4, ntask)):
            b = t % 4
            gd[b] = pltpu.async_copy(msg.at[pl.ds(base + w * ept + t * 64, 64)],
                                     bufs[b], gsems[b])
        for t in range(ntask):
            b = t % 4
            gd[b].wait()
            _fill_idx(idxs[b], dstb, t * 64, 4)
            sd[b] = pltpu.async_copy(bufs[b], shared.at[idxs[b]], ssems[b], add=True)
            nt = t + 4
            if nt < ntask:
                sd[b].wait()
                sd[b] = None
                gd[b] = pltpu.async_copy(msg.at[pl.ds(base + w * ept + nt * 64, 64)],
                                         bufs[b], gsems[b])
        for b in range(4):
            if sd[b] is not None:
                sd[b].wait()
        plsc.subcore_barrier()
        pltpu.sync_copy(shared.at[pl.ds(sid * 64, 64)], agg.at[cid, pl.ds(sid * 64, 64)])

    f = functools.partial(
        pl.kernel,
        out_type=jax.ShapeDtypeStruct((NC, N, W), jnp.float32),
        mesh=_get_mesh(),
        compiler_params=pltpu.CompilerParams(needs_layout_passes=False, use_tc_tiling_on_sc=False),
        scratch_types=[
            pltpu.VMEM((ept,), jnp.int32),
        ] + [pltpu.VMEM((64,), jnp.int32)] * 4
          + [pltpu.VMEM((64, W), jnp.float32)] * 4
          + [pltpu.SemaphoreType.DMA] * 8
          + [pltpu.VMEM_SHARED((N, W), jnp.float32)],
    )
    return f(body)


def _silu(x):
    return x * (1.0 / (1.0 + jnp.exp(-x)))


def _dot(a, b):
    return jax.lax.dot_general(a, b, (((1,), (0,)), ((), ())),
                               preferred_element_type=jnp.float32)


def _dotT(a, b):
    return jax.lax.dot_general(a, b, (((0,), (0,)), ((), ())),
                               preferred_element_type=jnp.float32)


def _tc_prep_a_body(x, W_atom, b_atom, posp, batch_row, W1s0, W1d0,
                    s0_o, PQ0_o, posc_o):
    s0 = _dot(x[...], W_atom[...]) + b_atom[...]
    s0_o[...] = s0
    PQ0_o[...] = jnp.concatenate([_dot(s0, W1s0[...]), _dot(s0, W1d0[...])], axis=1)
    M = (batch_row[...] == lax.broadcasted_iota(jnp.int32, (NB, N), 0)).astype(jnp.float32)
    cnt_b = jnp.sum(M, axis=1, keepdims=True)
    pos_mean = _dot(M, posp[...]) / jnp.maximum(cnt_b, 1.0)
    posc = posp[...] - _dotT(M, pos_mean)
    posc_o[...] = jnp.concatenate(
        [posc, jnp.zeros((N, 112), jnp.float32)], axis=1)


def _tc_prep_a(x, W_atom, b_atom, posp, batch_row, W1s0, W1d0):
    return pl.pallas_call(
        _tc_prep_a_body,
        out_shape=(
            jax.ShapeDtypeStruct((N, SDIM), jnp.float32),
            jax.ShapeDtypeStruct((N, 128), jnp.float32),
            jax.ShapeDtypeStruct((N, 128), jnp.float32),
        ),
    )(x, W_atom, b_atom, posp, batch_row, W1s0, W1d0)


def _tc_prep_b_body(ea, W_bond, b_bond, out):
    i = pl.program_id(0)
    et = _dot(ea[...], W_bond[...]) + b_bond[...]
    et = jnp.where(i == 0, jnp.zeros_like(et), et)
    out[...] = jnp.concatenate([et, jnp.zeros((et.shape[0], 128 - EDIM), jnp.float32)], axis=1)


def _tc_prep_b(ea8, W_bond8, b_bond):
    nb = NTBL // 512
    return pl.pallas_call(
        _tc_prep_b_body,
        grid=(nb,),
        in_specs=[
            pl.BlockSpec((512, 8), lambda i: (jnp.maximum(i - 1, 0), 0)),
            pl.BlockSpec((8, EDIM), lambda i: (0, 0)),
            pl.BlockSpec((1, EDIM), lambda i: (0, 0)),
        ],
        out_specs=pl.BlockSpec((512, 128), lambda i: (i, 0)),
        out_shape=jax.ShapeDtypeStruct((NTBL, 128), jnp.float32),
    )(ea8, W_bond8, b_bond)


def _tc_geom_body(ps, pd, e, G_o, rn_o):
    psv = ps[...]
    pdv = pd[...]
    r = pdv - psv
    d2 = jnp.sum(r * r, axis=1, keepdims=True)
    a = jnp.sum(psv * pdv, axis=1, keepdims=True)
    d = jnp.sqrt(jnp.maximum(d2, 1e-6))
    rn_o[...] = r / d
    mus = (CUTOFF / (RBF - 1)) * lax.broadcasted_iota(jnp.int32, (1, RBF), 1).astype(jnp.float32)
    rb = jnp.exp(-GAMMA * (d - mus) ** 2)
    G_o[...] = jnp.concatenate(
        [rb, e[:, :EDIM], a, jnp.zeros((rb.shape[0], 128 - RBF - EDIM - 1), jnp.float32)], axis=1)


def _tc_geom(pos_src, pos_dst, e_cat):
    nb = ETOT // 512
    return pl.pallas_call(
        _tc_geom_body,
        grid=(nb,),
        in_specs=[
            pl.BlockSpec((512, 128), lambda i: (i, 0)),
            pl.BlockSpec((512, 128), lambda i: (i, 0)),
            pl.BlockSpec((512, 128), lambda i: (i, 0)),
        ],
        out_specs=(
            pl.BlockSpec((512, 128), lambda i: (i, 0)),
            pl.BlockSpec((512, 128), lambda i: (i, 0)),
        ),
        out_shape=(
            jax.ShapeDtypeStruct((ETOT, 128), jnp.float32),
            jax.ShapeDtypeStruct((ETOT, 128), jnp.float32),
        ),
    )(pos_src, pos_dst, e_cat)


def _tc_et_body(G, W1g, b1, out):
    g = G[...]
    for l in range(L):
        out[l, :, :] = _dot(g, W1g[l]) + b1[l][None, :]


def _tc_et(G, W1g_pad, b1):
    nb = ETOT // 512
    return pl.pallas_call(
        _tc_et_body,
        grid=(nb,),
        in_specs=[
            pl.BlockSpec((512, 128), lambda i: (i, 0)),
            pl.BlockSpec((L, 128, MH), lambda i: (0, 0, 0)),
            pl.BlockSpec((L, MH), lambda i: (0, 0)),
        ],
        out_specs=pl.BlockSpec((L, 512, MH), lambda i: (0, i, 0)),
        out_shape=jax.ShapeDtypeStruct((L, ETOT, MH), jnp.float32),
    )(G, W1g_pad, b1)


def _make_tc_b(E, base, l):
    nb = E // 512

    def body(ET, PS, QD, rn, out):
        u = _silu(ET[0] + PS[:, :MH] + QD[:, MH:])
        rnv = rn[...]
        out[...] = jnp.concatenate(
            [u, u * rnv[:, 0:1], u * rnv[:, 1:2], u * rnv[:, 2:3]], axis=1)

    return pl.pallas_call(
        body,
        grid=(nb,),
        in_specs=[
            pl.BlockSpec((1, 512, MH), lambda i: (l, base // 512 + i, 0)),
            pl.BlockSpec((512, 128), lambda i: (i, 0)),
            pl.BlockSpec((512, 128), lambda i: (i, 0)),
            pl.BlockSpec((512, 128), lambda i: (base // 512 + i, 0)),
        ],
        out_specs=pl.BlockSpec((512, 4 * MH), lambda i: (i, 0)),
        out_shape=jax.ShapeDtypeStruct((E, 4 * MH), jnp.float32),
    )


def _tc_d1_body(s, v, aggL, cntl, Rl, W2s, W2v, b2s, b2v, W1s, W1d,
                s_mid_o, v_mid_o, PQ_o):
    U = aggL[0] + aggL[1]
    cnt = jnp.maximum(cntl[0, :, 0:1] + cntl[1, :, 0:1], 1.0)
    s_mid = s[...] + _dot(U[:, :MH], W2s[...]) + cnt * b2s[...]
    vparts = []
    for k in range(3):
        Rk = Rl[0, :, k:k + 1] + Rl[1, :, k:k + 1]
        vparts.append((_dot(U[:, MH * (k + 1):MH * (k + 2)], W2v[...]) + Rk * b2v[...]) / cnt)
    v_mid_o[...] = v[...] + jnp.concatenate(vparts, axis=1)
    s_mid_o[...] = s_mid
    PQ_o[...] = jnp.concatenate([_dot(s_mid, W1s[...]), _dot(s_mid, W1d[...])], axis=1)


def _tc_d1(s, v, aggL, cntl, Rl, W2s, W2v, b2s, b2v, W1s, W1d):
    return pl.pallas_call(
        _tc_d1_body,
        out_shape=(
            jax.ShapeDtypeStruct((N, SDIM), jnp.float32),
            jax.ShapeDtypeStruct((N, 3 * VDIM), jnp.float32),
            jax.ShapeDtypeStruct((N, 128), jnp.float32),
        ),
    )(s, v, aggL, cntl, Rl, W2s, W2v, b2s, b2v, W1s, W1d)


def _tc_d2_body(s_mid, v_mid, aggG, cntg, Rg, W2s, W2v, b2s, b2v, Wvl, W1s, W1d,
                s_o, v_o, PQ_o):
    U = aggG[0] + aggG[1]
    cnt = jnp.maximum(cntg[0, :, 0:1] + cntg[1, :, 0:1], 1.0)
    s2 = s_mid[...] + _dot(U[:, :MH], W2s[...]) + cnt * b2s[...]
    vparts = []
    for k in range(3):
        Rk = Rg[0, :, k:k + 1] + Rg[1, :, k:k + 1]
        vparts.append((_dot(U[:, MH * (k + 1):MH * (k + 2)], W2v[...]) + Rk * b2v[...]) / cnt)
    v_new = v_mid[...] + jnp.concatenate(vparts, axis=1)
    v_o[...] = v_new
    vn = jnp.sqrt(v_new[:, :VDIM] ** 2 + v_new[:, VDIM:2 * VDIM] ** 2
                  + v_new[:, 2 * VDIM:] ** 2 + 1e-6)
    sp = s2 + _dot(vn, Wvl[...])
    m = jnp.mean(sp, axis=1, keepdims=True)
    c = sp - m
    var = jnp.mean(c * c, axis=1, keepdims=True)
    s_new = c / jnp.sqrt(var + 1e-5)
    s_o[...] = s_new
    PQ_o[...] = jnp.concatenate([_dot(s_new, W1s[...]), _dot(s_new, W1d[...])], axis=1)


def _tc_d2(s_mid, v_mid, aggG, cntg, Rg, W2s, W2v, b2s, b2v, Wvl, W1s, W1d):
    return pl.pallas_call(
        _tc_d2_body,
        out_shape=(
            jax.ShapeDtypeStruct((N, SDIM), jnp.float32),
            jax.ShapeDtypeStruct((N, 3 * VDIM), jnp.float32),
            jax.ShapeDtypeStruct((N, 128), jnp.float32),
        ),
    )(s_mid, v_mid, aggG, cntg, Rg, W2s, W2v, b2s, b2v, Wvl, W1s, W1d)


def _tc_readout_body(s, batch_col, W_lat, b_lat, Wn1, bn1, Wn2, bn2,
                     Wg1, bg1, Wg2, bg2, pooled_o):
    out = _dot(s[...], W_lat[...]) + b_lat[...]
    g1 = _silu(_dot(out, Wg1[...]) + bg1[...])
    gate = _dot(g1, Wg2[...]) + bg2[...]
    nd = _silu(_dot(out, Wn1[...]) + bn1[...])
    nd = _dot(nd, Wn2[...]) + bn2[...]
    MT = (batch_col[:, 0:1] == lax.broadcasted_iota(jnp.int32, (N, NB), 1))
    MTf = MT.astype(jnp.float32)
    masked = jnp.where(MT, jnp.broadcast_to(gate, (N, NB)), -1e30)
    gmax = jnp.max(masked, axis=0, keepdims=True)
    gmax_pn = jax.lax.dot_general(MTf, gmax, (((1,), (1,)), ((), ())),
                                  preferred_element_type=jnp.float32)
    ge = jnp.exp(gate - gmax_pn)
    gden = _dotT(MTf, ge)
    gden_pn = _dot(MTf, gden)
    gate_n = ge / jnp.maximum(gden_pn, 1e-16)
    pooled_o[...] = _dotT(MTf, gate_n * nd)


def _tc_readout(s, batch_col, W_lat, b_lat, Wn1, bn1, Wn2, bn2, Wg1, bg1, Wg2, bg2):
    return pl.pallas_call(
        _tc_readout_body,
        out_shape=jax.ShapeDtypeStruct((NB, LAT), jnp.float32),
    )(s, batch_col, W_lat, b_lat, Wn1, bn1, Wn2, bn2, Wg1, bg1, Wg2, bg2)


@jax.jit
def kernel(x, pos, edge_index_local, edge_index_global, edge_attr_global, batch,
           W_atom, b_atom, W_bond, b_bond, W1, b1, W2, b2, Wv, W_lat, b_lat,
           Wn1, bn1, Wn2, bn2, Wg1, bg1, Wg2, bg2):
    eil = edge_index_local.astype(jnp.int32)
    eig = edge_index_global.astype(jnp.int32)
    batch_i = batch.astype(jnp.int32)
    posp = jnp.pad(pos, ((0, 0), (0, 16 - 3)))
    batch_row = batch_i.reshape(1, N)
    batch_col = jnp.broadcast_to(batch_i.reshape(N, 1), (N, 8))
    ea8 = jnp.pad(edge_attr_global, ((0, 0), (0, 8 - FB)))
    W_bond8 = jnp.pad(W_bond, ((0, 8 - FB), (0, 0)))

    W1s = W1[:, :SDIM, :]
    W1d = W1[:, SDIM:2 * SDIM, :]
    W1g_pad = jnp.zeros((L, 128, MH), jnp.float32)
    W1g_pad = W1g_pad.at[:, :RBF, :].set(W1[:, 2 * SDIM:2 * SDIM + RBF, :])
    W1g_pad = W1g_pad.at[:, RBF:RBF + EDIM, :].set(W1[:, 2 * SDIM + RBF:2 * SDIM + RBF + EDIM, :])
    W1g_pad = W1g_pad.at[:, RBF + EDIM, :].set(W1[:, 2 * SDIM + RBF + EDIM, :])

    s0, PQ, posc = _tc_prep_a(x, W_atom, b_atom.reshape(1, SDIM), posp,
                                batch_row, W1s[0], W1d[0])
    e_table = _tc_prep_b(ea8, W_bond8, b_bond.reshape(1, EDIM))

    idmap, pos_src, pos_dst, cntl_part, cntg_part = _sc1a(eig, eil, posc)
    e_l = _sc1b(idmap, eil, e_table)

    e_cat = jnp.concatenate([e_l, e_table[TBL_OFF:]], axis=0)
    G, rn128 = _tc_geom(pos_src, pos_dst, e_cat)
    ET_all = _tc_et(G, W1g_pad, b1)

    sc2_l = _make_sc2(EL)
    sc2_g = _make_sc2(EG)
    sc3_l = _make_sc3(EL, 4 * MH, 0)
    sc3_g = _make_sc3(EG, 4 * MH, 0)
    z256 = jnp.zeros((64, 4 * MH), jnp.float32)
    z128 = jnp.zeros((64, 128), jnp.float32)
    Rl_part = _make_sc3(EL, 128, 0)(rn128, eil, z128)
    Rg_part = _make_sc3(EG, 128, EL)(rn128, eig, z128)

    W2s = W2[:, :, :SDIM]
    W2v = W2[:, :, SDIM:]
    b2s = b2[:, :SDIM]
    b2v = b2[:, SDIM:]

    s = s0
    v = jnp.zeros((N, 3 * VDIM), jnp.float32)
    for l in range(L):
        PS, QD = sc2_l(eil, PQ)
        msg = _make_tc_b(EL, 0, l)(ET_all, PS, QD, rn128)
        aggL = sc3_l(msg, eil, z256)
        s, v, PQ = _tc_d1(s, v, aggL, cntl_part, Rl_part, W2s[l], W2v[l],
                          b2s[l].reshape(1, SDIM), b2v[l].reshape(1, VDIM),
                          W1s[l], W1d[l])
        PS, QD = sc2_g(eig, PQ)
        msg = _make_tc_b(EG, EL, l)(ET_all, PS, QD, rn128)
        aggG = sc3_g(msg, eig, z256)
        ln = min(l + 1, L - 1)
        s, v, PQ = _tc_d2(s, v, aggG, cntg_part, Rg_part, W2s[l], W2v[l],
                          b2s[l].reshape(1, SDIM), b2v[l].reshape(1, VDIM),
                          Wv[l], W1s[ln], W1d[ln])

    return _tc_readout(s, batch_col, W_lat, b_lat.reshape(1, LAT),
                       Wn1, bn1.reshape(1, LAT), Wn2, bn2.reshape(1, LAT),
                       Wg1, bg1.reshape(1, LAT), Wg2, bg2.reshape(1, 1))


# speedup vs baseline: 10.8420x; 1.0068x over previous
import functools
import jax
import jax.numpy as jnp
from jax import lax
from jax.experimental import pallas as pl
from jax.experimental.pallas import tpu as pltpu
from jax.experimental.pallas import tpu_sc as plsc

N = 1024
FA = 16
FB = 5
EL = 16384
EG = 32768
NB = 32
SDIM = 256
VDIM = 64
EDIM = 32
RBF = 32
L = 5
LAT = 128
MH = 64
CUTOFF = 7.5
MOUT = SDIM + VDIM

ETOT = EL + EG
NW = 32
NC = 2
TBL_OFF = 512
NTBL = EG + TBL_OFF
GAMMA = (RBF / CUTOFF) ** 2

@functools.cache
def _get_mesh():
    return plsc.VectorSubcoreMesh(core_axis_name="c", subcore_axis_name="s")


def _wid():
    return lax.axis_index("s") * NC + lax.axis_index("c")


def _fill_idx(dst_buf, src_buf, src_base, nvec):
    def body(i, _):
        dst_buf[pl.ds(i * 16, 16)] = src_buf[pl.ds(src_base + i * 16, 16)]
        return 0
    lax.fori_loop(0, nvec, body, 0)


def _sc1a_body(eig, eil, posc, zi, zf, onesf, idmap, pos_src, pos_dst, cntl_part, cntg_part,
               idchunk, srcbA, dstbA, srcbB, dstbB, srcl, dstl, srcg, dstg,
               gbuf0, gbuf1, ones_b, idxC0, idxC1, idx2,
               cA, cB, g0, g1, w0, w1, sc0, sc1,
               shared_l, shared_g):
    w = _wid()
    cid = lax.axis_index("c")
    sid = lax.axis_index("s")
    NKT = N * N // NW
    lo = w * NKT

    zdesc = pltpu.async_copy(zi, idchunk, cA)
    odesc = pltpu.async_copy(onesf, ones_b, cB)
    zdesc.wait()
    odesc.wait()

    iota16 = lax.iota(jnp.int32, 16)
    CH = 2048
    nch = EG // CH
    sb = [srcbA, srcbB]
    db = [dstbA, dstbB]
    cs = [cA, cB]
    descs = [None, None]

    def start_chunk(ci, b):
        d1 = pltpu.async_copy(eig.at[0, pl.ds(ci * CH, CH)], sb[b], cs[b])
        pltpu.async_copy(eig.at[1, pl.ds(ci * CH, CH)], db[b], cs[b])
        return d1

    descs[0] = start_chunk(0, 0)
    for ci in range(nch):
        b = ci % 2
        if ci + 1 < nch:
            descs[(ci + 1) % 2] = start_chunk(ci + 1, (ci + 1) % 2)
        descs[b].wait()
        pltpu.make_async_copy(eig.at[1, pl.ds(ci * CH, CH)], db[b], cs[b]).wait()
        def vec_body(j, _, b=b, ci=ci):
            s16 = sb[b][pl.ds(j * 16, 16)]
            d16 = db[b][pl.ds(j * 16, 16)]
            k = s16 * N + d16 - lo
            ids = ci * CH + j * 16 + TBL_OFF + iota16
            m = (k >= 0) & (k < NKT)
            kc = jnp.where(m, k, 0)
            plsc.store_scatter(idchunk, [lax.shift_right_logical(kc, 7), kc & 127],
                               ids, mask=m)
            return 0
        lax.fori_loop(0, CH // 16, vec_body, 0)
    pltpu.sync_copy(idchunk, idmap.at[pl.ds(w * (NKT // 128), NKT // 128)])

    lpt = EL // NW
    gpt = EG // NW
    pltpu.sync_copy(eil.at[0, pl.ds(w * lpt, lpt)], srcl)
    pltpu.sync_copy(eil.at[1, pl.ds(w * lpt, lpt)], dstl)
    pltpu.sync_copy(eig.at[0, pl.ds(w * gpt, gpt)], srcg)
    pltpu.sync_copy(eig.at[1, pl.ds(w * gpt, gpt)], dstg)

    tasks = []
    for j in range(lpt // 128):
        tasks.append((srcl, j * 128, pos_src, w * lpt + j * 128))
        tasks.append((dstl, j * 128, pos_dst, w * lpt + j * 128))
    for j in range(gpt // 128):
        tasks.append((srcg, j * 128, pos_src, EL + w * gpt + j * 128))
        tasks.append((dstg, j * 128, pos_dst, EL + w * gpt + j * 128))
    ntask = len(tasks)
    for t, (buf, bas, _, _2) in enumerate(tasks):
        def fb(i, _, t=t, buf=buf, bas=bas):
            idx2[t, pl.ds(i * 16, 16)] = buf[pl.ds(bas + i * 16, 16)]
            return 0
        lax.fori_loop(0, 8, fb, 0)
    gbufs = [gbuf0, gbuf1]
    gsems = [g0, g1]
    wsems = [w0, w1]
    gd = [None] * 2
    wd = [None] * 2
    for t in range(min(2, ntask)):
        b = t % 2
        gd[b] = pltpu.async_copy(posc.at[idx2.at[t]], gbufs[b], gsems[b])
    for t in range(ntask):
        b = t % 2
        gd[b].wait()
        out, off = tasks[t][2], tasks[t][3]
        wd[b] = pltpu.async_copy(gbufs[b], out.at[pl.ds(off, 128)], wsems[b])
        nt = t + 2
        if nt < ntask:
            wd[b].wait()
            wd[b] = None
            gd[b] = pltpu.async_copy(posc.at[idx2.at[nt]], gbufs[b], gsems[b])
    for b in range(2):
        if wd[b] is not None:
            wd[b].wait()

    pltpu.sync_copy(zf.at[pl.ds(0, 64)], shared_l.at[pl.ds(sid * 64, 64)])
    pltpu.sync_copy(zf.at[pl.ds(64, 64)], shared_g.at[pl.ds(sid * 64, 64)])
    plsc.subcore_barrier()

    ctasks = []
    for j in range(lpt // 128):
        ctasks.append((dstl, j * 128, shared_l))
    for j in range(gpt // 128):
        ctasks.append((dstg, j * 128, shared_g))
    idxCs = [idxC0, idxC1]
    scs = [sc0, sc1]
    sd = [None, None]
    for t, (buf, bas, shared) in enumerate(ctasks):
        b = t % 2
        if sd[b] is not None:
            sd[b].wait()
        _fill_idx(idxCs[b], buf, bas, 8)
        sd[b] = pltpu.async_copy(ones_b, shared.at[idxCs[b]], scs[b], add=True)
    for b in range(2):
        if sd[b] is not None:
            sd[b].wait()
    plsc.subcore_barrier()

    pltpu.sync_copy(shared_l.at[pl.ds(sid * 64, 64)], cntl_part.at[cid, pl.ds(sid * 64, 64)])
    pltpu.sync_copy(shared_g.at[pl.ds(sid * 64, 64)], cntg_part.at[cid, pl.ds(sid * 64, 64)])


def _sc1a(eig, eil, posc, zi, zf, onesf):
    f = functools.partial(
        pl.kernel,
        out_type=(
            jax.ShapeDtypeStruct((N * N // 128, 128), jnp.int32),
            jax.ShapeDtypeStruct((ETOT, 128), jnp.float32),
            jax.ShapeDtypeStruct((ETOT, 128), jnp.float32),
            jax.ShapeDtypeStruct((NC, N, 128), jnp.float32),
            jax.ShapeDtypeStruct((NC, N, 128), jnp.float32),
        ),
        mesh=_get_mesh(),
        compiler_params=pltpu.CompilerParams(needs_layout_passes=False, use_tc_tiling_on_sc=False),
        scratch_types=[
            pltpu.VMEM((N * N // NW // 128, 128), jnp.int32),
            pltpu.VMEM((2048,), jnp.int32),
            pltpu.VMEM((2048,), jnp.int32),
            pltpu.VMEM((2048,), jnp.int32),
            pltpu.VMEM((2048,), jnp.int32),
            pltpu.VMEM((EL // NW,), jnp.int32),
            pltpu.VMEM((EL // NW,), jnp.int32),
            pltpu.VMEM((EG // NW,), jnp.int32),
            pltpu.VMEM((EG // NW,), jnp.int32),
        ] + [pltpu.VMEM((128, 128), jnp.float32)] * 2 + [
            pltpu.VMEM((128, 128), jnp.float32),
            pltpu.VMEM((128,), jnp.int32),
            pltpu.VMEM((128,), jnp.int32),
            pltpu.VMEM((24, 128), jnp.int32),
        ] + [pltpu.SemaphoreType.DMA] * 8 + [
            pltpu.VMEM_SHARED((N, 128), jnp.float32),
            pltpu.VMEM_SHARED((N, 128), jnp.float32),
        ],
    )
    return f(_sc1a_body)(eig, eil, posc, zi, zf, onesf)


def _sc1b_body(idmap, eil, e_table, e_l, srcl, dstl, keyrow, keycol,
               idxA, idxB, lidA, lidB, rowsA, rowsB, erowsA, erowsB,
               gA, gB, tA, tB, wA, wB):
    w = _wid()
    lpt = EL // NW
    nj = lpt // 128
    iota16 = lax.iota(jnp.int32, 16)
    pltpu.sync_copy(eil.at[0, pl.ds(w * lpt, lpt)], srcl)
    pltpu.sync_copy(eil.at[1, pl.ds(w * lpt, lpt)], dstl)
    def kbody(i, _):
        key = srcl[pl.ds(i * 16, 16)] * N + dstl[pl.ds(i * 16, 16)]
        keyrow[pl.ds(i * 16, 16)] = lax.shift_right_logical(key, 7)
        keycol[pl.ds(i * 16, 16)] = key & 127
        return 0
    lax.fori_loop(0, lpt // 16, kbody, 0)

    idxs = [idxA, idxB]
    lids = [lidA, lidB]
    rows = [rowsA, rowsB]
    erows = [erowsA, erowsB]
    gs = [gA, gB]
    ts = [tA, tB]
    ws = [wA, wB]

    def start_g(j, b):
        _fill_idx(idxs[b], keyrow, j * 128, 8)
        return pltpu.async_copy(idmap.at[idxs[b]], rows[b], gs[b])

    gd = [None, None]
    wd = [None, None]
    gd[0] = start_g(0, 0)
    for j in range(nj):
        b = j % 2
        gd[b].wait()
        if j + 1 < nj:
            gd[(j + 1) % 2] = start_g(j + 1, (j + 1) % 2)
        for t in range(8):
            r16 = t * 16 + iota16
            c16 = keycol[pl.ds(j * 128 + t * 16, 16)]
            lids[b][pl.ds(t * 16, 16)] = plsc.load_gather(rows[b], [r16, c16])
        if wd[b] is not None:
            wd[b].wait()
            wd[b] = None
        pltpu.async_copy(e_table.at[lids[b]], erows[b], ts[b]).wait()
        wd[b] = pltpu.async_copy(erows[b], e_l.at[pl.ds(w * lpt + j * 128, 128)], ws[b])
    for b in range(2):
        if wd[b] is not None:
            wd[b].wait()


def _sc1b(idmap, eil, e_table):
    f = functools.partial(
        pl.kernel,
        out_type=jax.ShapeDtypeStruct((EL, 128), jnp.float32),
        mesh=_get_mesh(),
        compiler_params=pltpu.CompilerParams(needs_layout_passes=False, use_tc_tiling_on_sc=False),
        scratch_types=[
            pltpu.VMEM((EL // NW,), jnp.int32),
            pltpu.VMEM((EL // NW,), jnp.int32),
            pltpu.VMEM((EL // NW,), jnp.int32),
            pltpu.VMEM((EL // NW,), jnp.int32),
            pltpu.VMEM((128,), jnp.int32),
            pltpu.VMEM((128,), jnp.int32),
            pltpu.VMEM((128,), jnp.int32),
            pltpu.VMEM((128,), jnp.int32),
            pltpu.VMEM((128, 128), jnp.int32),
            pltpu.VMEM((128, 128), jnp.int32),
            pltpu.VMEM((128, 128), jnp.float32),
            pltpu.VMEM((128, 128), jnp.float32),
        ] + [pltpu.SemaphoreType.DMA] * 6,
    )
    return f(_sc1b_body)(idmap, eil, e_table)


def _make_sc2(E):
    ept = E // NW
    nj = ept // 128
    ntask = 2 * nj

    def body(ei, PQ, PS, QD, srcb, dstb, idx2,
             buf0, buf1, buf2, buf3, g0, g1, g2, g3, w0, w1, w2, w3):
        w = _wid()
        bufs = [buf0, buf1, buf2, buf3]
        gsems = [g0, g1, g2, g3]
        wsems = [w0, w1, w2, w3]
        pltpu.sync_copy(ei.at[0, pl.ds(w * ept, ept)], srcb)
        pltpu.sync_copy(ei.at[1, pl.ds(w * ept, ept)], dstb)
        for j in range(nj):
            def fb(i, _, j=j):
                idx2[j, pl.ds(i * 16, 16)] = srcb[pl.ds(j * 128 + i * 16, 16)]
                idx2[nj + j, pl.ds(i * 16, 16)] = dstb[pl.ds(j * 128 + i * 16, 16)]
                return 0
            lax.fori_loop(0, 8, fb, 0)

        def out_at(t):
            if t < nj:
                return PS.at[pl.ds(w * ept + t * 128, 128)]
            return QD.at[pl.ds(w * ept + (t - nj) * 128, 128)]

        gd = [None] * 4
        wd = [None] * 4
        for t in range(min(4, ntask)):
            b = t % 4
            gd[b] = pltpu.async_copy(PQ.at[idx2.at[t]], bufs[b], gsems[b])
        for t in range(ntask):
            b = t % 4
            gd[b].wait()
            wd[b] = pltpu.async_copy(bufs[b], out_at(t), wsems[b])
            nt = t + 4
            if nt < ntask:
                wd[b].wait()
                wd[b] = None
                gd[b] = pltpu.async_copy(PQ.at[idx2.at[nt]], bufs[b], gsems[b])
        for b in range(4):
            if wd[b] is not None:
                wd[b].wait()

    f = functools.partial(
        pl.kernel,
        out_type=(
            jax.ShapeDtypeStruct((E, 128), jnp.float32),
            jax.ShapeDtypeStruct((E, 128), jnp.float32),
        ),
        mesh=_get_mesh(),
        compiler_params=pltpu.CompilerParams(needs_layout_passes=False, use_tc_tiling_on_sc=False),
        scratch_types=[
            pltpu.VMEM((ept,), jnp.int32),
            pltpu.VMEM((ept,), jnp.int32),
            pltpu.VMEM((ntask, 128), jnp.int32),
        ] + [pltpu.VMEM((128, 128), jnp.float32)] * 4
          + [pltpu.SemaphoreType.DMA] * 8,
    )
    return f(body)


MSGW = SDIM + 3 * VDIM


def _make_sc3(E, W, base):
    ept = E // NW
    ntask = ept // 64

    def body(msg, ei, zrows, agg, dstb,
             idx0, idx1, idx2_, idx3, buf0, buf1, buf2, buf3,
             g0, g1, g2, g3, s0, s1, s2, s3, shared):
        w = _wid()
        cid = lax.axis_index("c")
        sid = lax.axis_index("s")
        bufs = [buf0, buf1, buf2, buf3]
        idxs = [idx0, idx1, idx2_, idx3]
        gsems = [g0, g1, g2, g3]
        ssems = [s0, s1, s2, s3]
        pltpu.sync_copy(zrows, shared.at[pl.ds(sid * 64, 64)])
        plsc.subcore_barrier()
        pltpu.sync_copy(ei.at[1, pl.ds(w * ept, ept)], dstb)

        gd = [None] * 4
        sd = [None] * 4
        for t in range(min(4, ntask)):
            b = t % 4
            gd[b] = pltpu.async_copy(msg.at[pl.ds(base + w * ept + t * 64, 64)],
                                     bufs[b], gsems[b])
        for t in range(ntask):
            b = t % 4
            gd[b].wait()
            _fill_idx(idxs[b], dstb, t * 64, 4)
            sd[b] = pltpu.async_copy(bufs[b], shared.at[idxs[b]], ssems[b], add=True)
            nt = t + 4
            if nt < ntask:
                sd[b].wait()
                sd[b] = None
                gd[b] = pltpu.async_copy(msg.at[pl.ds(base + w * ept + nt * 64, 64)],
                                         bufs[b], gsems[b])
        for b in range(4):
            if sd[b] is not None:
                sd[b].wait()
        plsc.subcore_barrier()
        pltpu.sync_copy(shared.at[pl.ds(sid * 64, 64)], agg.at[cid, pl.ds(sid * 64, 64)])

    f = functools.partial(
        pl.kernel,
        out_type=jax.ShapeDtypeStruct((NC, N, W), jnp.float32),
        mesh=_get_mesh(),
        compiler_params=pltpu.CompilerParams(needs_layout_passes=False, use_tc_tiling_on_sc=False),
        scratch_types=[
            pltpu.VMEM((ept,), jnp.int32),
        ] + [pltpu.VMEM((64,), jnp.int32)] * 4
          + [pltpu.VMEM((64, W), jnp.float32)] * 4
          + [pltpu.SemaphoreType.DMA] * 8
          + [pltpu.VMEM_SHARED((N, W), jnp.float32)],
    )
    return f(body)


def _silu(x):
    return x * (1.0 / (1.0 + jnp.exp(-x)))


def _dot(a, b):
    return jax.lax.dot_general(a, b, (((1,), (0,)), ((), ())),
                               preferred_element_type=jnp.float32)


def _dotT(a, b):
    return jax.lax.dot_general(a, b, (((0,), (0,)), ((), ())),
                               preferred_element_type=jnp.float32)


def _tc_prep_a_body(x, W_atom, b_atom, posp, batch_row, W1s0, W1d0,
                    s0_o, PQ0_o, posc_o):
    s0 = _dot(x[...], W_atom[...]) + b_atom[...]
    s0_o[...] = s0
    PQ0_o[...] = jnp.concatenate([_dot(s0, W1s0[...]), _dot(s0, W1d0[...])], axis=1)
    M = (batch_row[...] == lax.broadcasted_iota(jnp.int32, (NB, N), 0)).astype(jnp.float32)
    cnt_b = jnp.sum(M, axis=1, keepdims=True)
    pos_mean = _dot(M, posp[...]) / jnp.maximum(cnt_b, 1.0)
    posc = posp[...] - _dotT(M, pos_mean)
    posc_o[...] = jnp.concatenate(
        [posc, jnp.zeros((N, 112), jnp.float32)], axis=1)


def _tc_prep_a(x, W_atom, b_atom, posp, batch_row, W1s0, W1d0):
    return pl.pallas_call(
        _tc_prep_a_body,
        out_shape=(
            jax.ShapeDtypeStruct((N, SDIM), jnp.float32),
            jax.ShapeDtypeStruct((N, 128), jnp.float32),
            jax.ShapeDtypeStruct((N, 128), jnp.float32),
        ),
    )(x, W_atom, b_atom, posp, batch_row, W1s0, W1d0)


def _tc_prep_b_body(ea, W_bond, b_bond, out):
    i = pl.program_id(0)
    et = _dot(ea[...], W_bond[...]) + b_bond[...]
    et = jnp.where(i == 0, jnp.zeros_like(et), et)
    out[...] = jnp.concatenate([et, jnp.zeros((et.shape[0], 128 - EDIM), jnp.float32)], axis=1)


def _tc_prep_b(ea8, W_bond8, b_bond):
    nb = NTBL // 512
    return pl.pallas_call(
        _tc_prep_b_body,
        grid=(nb,),
        in_specs=[
            pl.BlockSpec((512, 8), lambda i: (jnp.maximum(i - 1, 0), 0)),
            pl.BlockSpec((8, EDIM), lambda i: (0, 0)),
            pl.BlockSpec((1, EDIM), lambda i: (0, 0)),
        ],
        out_specs=pl.BlockSpec((512, 128), lambda i: (i, 0)),
        out_shape=jax.ShapeDtypeStruct((NTBL, 128), jnp.float32),
    )(ea8, W_bond8, b_bond)


def _tc_geom_body(ps, pd, e, G_o, rn_o):
    psv = ps[...]
    pdv = pd[...]
    r = pdv - psv
    d2 = jnp.sum(r * r, axis=1, keepdims=True)
    a = jnp.sum(psv * pdv, axis=1, keepdims=True)
    d = jnp.sqrt(jnp.maximum(d2, 1e-6))
    rn_o[...] = r / d
    mus = (CUTOFF / (RBF - 1)) * lax.broadcasted_iota(jnp.int32, (1, RBF), 1).astype(jnp.float32)
    rb = jnp.exp(-GAMMA * (d - mus) ** 2)
    G_o[...] = jnp.concatenate(
        [rb, e[:, :EDIM], a, jnp.zeros((rb.shape[0], 128 - RBF - EDIM - 1), jnp.float32)], axis=1)


def _tc_geom(pos_src, pos_dst, e_cat):
    nb = ETOT // 512
    return pl.pallas_call(
        _tc_geom_body,
        grid=(nb,),
        in_specs=[
            pl.BlockSpec((512, 128), lambda i: (i, 0)),
            pl.BlockSpec((512, 128), lambda i: (i, 0)),
            pl.BlockSpec((512, 128), lambda i: (i, 0)),
        ],
        out_specs=(
            pl.BlockSpec((512, 128), lambda i: (i, 0)),
            pl.BlockSpec((512, 128), lambda i: (i, 0)),
        ),
        out_shape=(
            jax.ShapeDtypeStruct((ETOT, 128), jnp.float32),
            jax.ShapeDtypeStruct((ETOT, 128), jnp.float32),
        ),
    )(pos_src, pos_dst, e_cat)


def _tc_et_body(G, W1g, b1, out):
    g = G[...]
    for l in range(L):
        out[l, :, :] = _dot(g, W1g[l]) + b1[l][None, :]


def _tc_et(G, W1g_pad, b1):
    nb = ETOT // 512
    return pl.pallas_call(
        _tc_et_body,
        grid=(nb,),
        in_specs=[
            pl.BlockSpec((512, 128), lambda i: (i, 0)),
            pl.BlockSpec((L, 128, MH), lambda i: (0, 0, 0)),
            pl.BlockSpec((L, MH), lambda i: (0, 0)),
        ],
        out_specs=pl.BlockSpec((L, 512, MH), lambda i: (0, i, 0)),
        out_shape=jax.ShapeDtypeStruct((L, ETOT, MH), jnp.float32),
    )(G, W1g_pad, b1)


def _make_tc_b(E, base, l):
    nb = E // 512

    def body(ET, PS, QD, rn, out):
        u = _silu(ET[0] + PS[:, :MH] + QD[:, MH:])
        rnv = rn[...]
        out[...] = jnp.concatenate(
            [u, u * rnv[:, 0:1], u * rnv[:, 1:2], u * rnv[:, 2:3]], axis=1)

    return pl.pallas_call(
        body,
        grid=(nb,),
        in_specs=[
            pl.BlockSpec((1, 512, MH), lambda i: (l, base // 512 + i, 0)),
            pl.BlockSpec((512, 128), lambda i: (i, 0)),
            pl.BlockSpec((512, 128), lambda i: (i, 0)),
            pl.BlockSpec((512, 128), lambda i: (base // 512 + i, 0)),
        ],
        out_specs=pl.BlockSpec((512, 4 * MH), lambda i: (i, 0)),
        out_shape=jax.ShapeDtypeStruct((E, 4 * MH), jnp.float32),
    )


def _tc_d1_body(s, v, aggL, cntl, Rl, W2s, W2v, b2s, b2v, W1s, W1d,
                s_mid_o, v_mid_o, PQ_o):
    U = aggL[0] + aggL[1]
    cnt = jnp.maximum(cntl[0, :, 0:1] + cntl[1, :, 0:1], 1.0)
    s_mid = s[...] + _dot(U[:, :MH], W2s[...]) + cnt * b2s[...]
    vparts = []
    for k in range(3):
        Rk = Rl[0, :, k:k + 1] + Rl[1, :, k:k + 1]
        vparts.append((_dot(U[:, MH * (k + 1):MH * (k + 2)], W2v[...]) + Rk * b2v[...]) / cnt)
    v_mid_o[...] = v[...] + jnp.concatenate(vparts, axis=1)
    s_mid_o[...] = s_mid
    PQ_o[...] = jnp.concatenate([_dot(s_mid, W1s[...]), _dot(s_mid, W1d[...])], axis=1)


def _tc_d1(s, v, aggL, cntl, Rl, W2s, W2v, b2s, b2v, W1s, W1d):
    return pl.pallas_call(
        _tc_d1_body,
        out_shape=(
            jax.ShapeDtypeStruct((N, SDIM), jnp.float32),
            jax.ShapeDtypeStruct((N, 3 * VDIM), jnp.float32),
            jax.ShapeDtypeStruct((N, 128), jnp.float32),
        ),
    )(s, v, aggL, cntl, Rl, W2s, W2v, b2s, b2v, W1s, W1d)


def _tc_d2_body(s_mid, v_mid, aggG, cntg, Rg, W2s, W2v, b2s, b2v, Wvl, W1s, W1d,
                s_o, v_o, PQ_o):
    U = aggG[0] + aggG[1]
    cnt = jnp.maximum(cntg[0, :, 0:1] + cntg[1, :, 0:1], 1.0)
    s2 = s_mid[...] + _dot(U[:, :MH], W2s[...]) + cnt * b2s[...]
    vparts = []
    for k in range(3):
        Rk = Rg[0, :, k:k + 1] + Rg[1, :, k:k + 1]
        vparts.append((_dot(U[:, MH * (k + 1):MH * (k + 2)], W2v[...]) + Rk * b2v[...]) / cnt)
    v_new = v_mid[...] + jnp.concatenate(vparts, axis=1)
    v_o[...] = v_new
    vn = jnp.sqrt(v_new[:, :VDIM] ** 2 + v_new[:, VDIM:2 * VDIM] ** 2
                  + v_new[:, 2 * VDIM:] ** 2 + 1e-6)
    sp = s2 + _dot(vn, Wvl[...])
    m = jnp.mean(sp, axis=1, keepdims=True)
    c = sp - m
    var = jnp.mean(c * c, axis=1, keepdims=True)
    s_new = c / jnp.sqrt(var + 1e-5)
    s_o[...] = s_new
    PQ_o[...] = jnp.concatenate([_dot(s_new, W1s[...]), _dot(s_new, W1d[...])], axis=1)


def _tc_d2(s_mid, v_mid, aggG, cntg, Rg, W2s, W2v, b2s, b2v, Wvl, W1s, W1d):
    return pl.pallas_call(
        _tc_d2_body,
        out_shape=(
            jax.ShapeDtypeStruct((N, SDIM), jnp.float32),
            jax.ShapeDtypeStruct((N, 3 * VDIM), jnp.float32),
            jax.ShapeDtypeStruct((N, 128), jnp.float32),
        ),
    )(s_mid, v_mid, aggG, cntg, Rg, W2s, W2v, b2s, b2v, Wvl, W1s, W1d)


def _tc_readout_body(s, batch_col, W_lat, b_lat, Wn1, bn1, Wn2, bn2,
                     Wg1, bg1, Wg2, bg2, pooled_o):
    out = _dot(s[...], W_lat[...]) + b_lat[...]
    g1 = _silu(_dot(out, Wg1[...]) + bg1[...])
    gate = _dot(g1, Wg2[...]) + bg2[...]
    nd = _silu(_dot(out, Wn1[...]) + bn1[...])
    nd = _dot(nd, Wn2[...]) + bn2[...]
    MT = (batch_col[:, 0:1] == lax.broadcasted_iota(jnp.int32, (N, NB), 1))
    MTf = MT.astype(jnp.float32)
    masked = jnp.where(MT, jnp.broadcast_to(gate, (N, NB)), -1e30)
    gmax = jnp.max(masked, axis=0, keepdims=True)
    gmax_pn = jax.lax.dot_general(MTf, gmax, (((1,), (1,)), ((), ())),
                                  preferred_element_type=jnp.float32)
    ge = jnp.exp(gate - gmax_pn)
    gden = _dotT(MTf, ge)
    gden_pn = _dot(MTf, gden)
    gate_n = ge / jnp.maximum(gden_pn, 1e-16)
    pooled_o[...] = _dotT(MTf, gate_n * nd)


def _tc_readout(s, batch_col, W_lat, b_lat, Wn1, bn1, Wn2, bn2, Wg1, bg1, Wg2, bg2):
    return pl.pallas_call(
        _tc_readout_body,
        out_shape=jax.ShapeDtypeStruct((NB, LAT), jnp.float32),
    )(s, batch_col, W_lat, b_lat, Wn1, bn1, Wn2, bn2, Wg1, bg1, Wg2, bg2)


@jax.jit
def kernel(x, pos, edge_index_local, edge_index_global, edge_attr_global, batch,
           W_atom, b_atom, W_bond, b_bond, W1, b1, W2, b2, Wv, W_lat, b_lat,
           Wn1, bn1, Wn2, bn2, Wg1, bg1, Wg2, bg2):
    eil = edge_index_local.astype(jnp.int32)
    eig = edge_index_global.astype(jnp.int32)
    batch_i = batch.astype(jnp.int32)
    posp = jnp.pad(pos, ((0, 0), (0, 16 - 3)))
    batch_row = batch_i.reshape(1, N)
    batch_col = jnp.broadcast_to(batch_i.reshape(N, 1), (N, 8))
    ea8 = jnp.pad(edge_attr_global, ((0, 0), (0, 8 - FB)))
    W_bond8 = jnp.pad(W_bond, ((0, 8 - FB), (0, 0)))

    W1s = W1[:, :SDIM, :]
    W1d = W1[:, SDIM:2 * SDIM, :]
    W1g_pad = jnp.zeros((L, 128, MH), jnp.float32)
    W1g_pad = W1g_pad.at[:, :RBF, :].set(W1[:, 2 * SDIM:2 * SDIM + RBF, :])
    W1g_pad = W1g_pad.at[:, RBF:RBF + EDIM, :].set(W1[:, 2 * SDIM + RBF:2 * SDIM + RBF + EDIM, :])
    W1g_pad = W1g_pad.at[:, RBF + EDIM, :].set(W1[:, 2 * SDIM + RBF + EDIM, :])

    s0, PQ, posc = _tc_prep_a(x, W_atom, b_atom.reshape(1, SDIM), posp,
                                batch_row, W1s[0], W1d[0])
    e_table = _tc_prep_b(ea8, W_bond8, b_bond.reshape(1, EDIM))

    zi = jnp.zeros((N * N // NW // 128, 128), jnp.int32)
    zf = jnp.zeros((128, 128), jnp.float32)
    onesf = jnp.ones((128, 128), jnp.float32)
    idmap, pos_src, pos_dst, cntl_part, cntg_part = _sc1a(eig, eil, posc, zi, zf, onesf)
    e_l = _sc1b(idmap, eil, e_table)

    e_cat = jnp.concatenate([e_l, e_table[TBL_OFF:]], axis=0)
    G, rn128 = _tc_geom(pos_src, pos_dst, e_cat)
    ET_all = _tc_et(G, W1g_pad, b1)

    sc2_l = _make_sc2(EL)
    sc2_g = _make_sc2(EG)
    sc3_l = _make_sc3(EL, 4 * MH, 0)
    sc3_g = _make_sc3(EG, 4 * MH, 0)
    z256 = jnp.zeros((64, 4 * MH), jnp.float32)
    z128 = jnp.zeros((64, 128), jnp.float32)
    Rl_part = _make_sc3(EL, 128, 0)(rn128, eil, z128)
    Rg_part = _make_sc3(EG, 128, EL)(rn128, eig, z128)

    W2s = W2[:, :, :SDIM]
    W2v = W2[:, :, SDIM:]
    b2s = b2[:, :SDIM]
    b2v = b2[:, SDIM:]

    s = s0
    v = jnp.zeros((N, 3 * VDIM), jnp.float32)
    for l in range(L):
        PS, QD = sc2_l(eil, PQ)
        msg = _make_tc_b(EL, 0, l)(ET_all, PS, QD, rn128)
        aggL = sc3_l(msg, eil, z256)
        s, v, PQ = _tc_d1(s, v, aggL, cntl_part, Rl_part, W2s[l], W2v[l],
                          b2s[l].reshape(1, SDIM), b2v[l].reshape(1, VDIM),
                          W1s[l], W1d[l])
        PS, QD = sc2_g(eig, PQ)
        msg = _make_tc_b(EG, EL, l)(ET_all, PS, QD, rn128)
        aggG = sc3_g(msg, eig, z256)
        ln = min(l + 1, L - 1)
        s, v, PQ = _tc_d2(s, v, aggG, cntg_part, Rg_part, W2s[l], W2v[l],
                          b2s[l].reshape(1, SDIM), b2v[l].reshape(1, VDIM),
                          Wv[l], W1s[ln], W1d[ln])

    return _tc_readout(s, batch_col, W_lat, b_lat.reshape(1, LAT),
                       Wn1, bn1.reshape(1, LAT), Wn2, bn2.reshape(1, LAT),
                       Wg1, bg1.reshape(1, LAT), Wg2, bg2.reshape(1, 1))


# speedup vs baseline: 10.8508x; 1.0008x over previous
import functools
import jax
import jax.numpy as jnp
from jax import lax
from jax.experimental import pallas as pl
from jax.experimental.pallas import tpu as pltpu
from jax.experimental.pallas import tpu_sc as plsc

N = 1024
FA = 16
FB = 5
EL = 16384
EG = 32768
NB = 32
SDIM = 256
VDIM = 64
EDIM = 32
RBF = 32
L = 5
LAT = 128
MH = 64
CUTOFF = 7.5
MOUT = SDIM + VDIM

ETOT = EL + EG
NW = 32
NC = 2
TBL_OFF = 512
NTBL = EG + TBL_OFF
GAMMA = (RBF / CUTOFF) ** 2

@functools.cache
def _get_mesh():
    return plsc.VectorSubcoreMesh(core_axis_name="c", subcore_axis_name="s")


def _wid():
    return lax.axis_index("s") * NC + lax.axis_index("c")


def _fill_idx(dst_buf, src_buf, src_base, nvec):
    @plsc.parallel_loop(0, nvec, unroll=nvec)
    def _(i):
        dst_buf[pl.ds(i * 16, 16)] = src_buf[pl.ds(src_base + i * 16, 16)]


def _sc1a_body(eig, eil, posc, zi, zf, onesf, idmap, pos_src, pos_dst, cntl_part, cntg_part,
               idchunk, srcbA, dstbA, srcbB, dstbB, srcl, dstl, srcg, dstg,
               gbuf0, gbuf1, ones_b, idxC0, idxC1, idx2,
               cA, cB, g0, g1, w0, w1, sc0, sc1,
               shared_l, shared_g):
    w = _wid()
    cid = lax.axis_index("c")
    sid = lax.axis_index("s")
    NKT = N * N // NW
    lo = w * NKT

    zdesc = pltpu.async_copy(zi, idchunk, cA)
    odesc = pltpu.async_copy(onesf, ones_b, cB)
    zdesc.wait()
    odesc.wait()

    iota16 = lax.iota(jnp.int32, 16)
    CH = 2048
    nch = EG // CH
    sb = [srcbA, srcbB]
    db = [dstbA, dstbB]
    cs = [cA, cB]
    descs = [None, None]

    def start_chunk(ci, b):
        d1 = pltpu.async_copy(eig.at[0, pl.ds(ci * CH, CH)], sb[b], cs[b])
        pltpu.async_copy(eig.at[1, pl.ds(ci * CH, CH)], db[b], cs[b])
        return d1

    descs[0] = start_chunk(0, 0)
    for ci in range(nch):
        b = ci % 2
        if ci + 1 < nch:
            descs[(ci + 1) % 2] = start_chunk(ci + 1, (ci + 1) % 2)
        descs[b].wait()
        pltpu.make_async_copy(eig.at[1, pl.ds(ci * CH, CH)], db[b], cs[b]).wait()
        @plsc.parallel_loop(0, CH // 16, unroll=8)
        def _(j, b=b, ci=ci):
            s16 = sb[b][pl.ds(j * 16, 16)]
            d16 = db[b][pl.ds(j * 16, 16)]
            k = s16 * N + d16 - lo
            ids = ci * CH + j * 16 + TBL_OFF + iota16
            m = (k >= 0) & (k < NKT)
            kc = jnp.where(m, k, 0)
            plsc.store_scatter(idchunk, [lax.shift_right_logical(kc, 7), kc & 127],
                               ids, mask=m)
    pltpu.sync_copy(idchunk, idmap.at[pl.ds(w * (NKT // 128), NKT // 128)])

    lpt = EL // NW
    gpt = EG // NW
    pltpu.sync_copy(eil.at[0, pl.ds(w * lpt, lpt)], srcl)
    pltpu.sync_copy(eil.at[1, pl.ds(w * lpt, lpt)], dstl)
    pltpu.sync_copy(eig.at[0, pl.ds(w * gpt, gpt)], srcg)
    pltpu.sync_copy(eig.at[1, pl.ds(w * gpt, gpt)], dstg)

    tasks = []
    for j in range(lpt // 128):
        tasks.append((srcl, j * 128, pos_src, w * lpt + j * 128))
        tasks.append((dstl, j * 128, pos_dst, w * lpt + j * 128))
    for j in range(gpt // 128):
        tasks.append((srcg, j * 128, pos_src, EL + w * gpt + j * 128))
        tasks.append((dstg, j * 128, pos_dst, EL + w * gpt + j * 128))
    ntask = len(tasks)
    for t, (buf, bas, _, _2) in enumerate(tasks):
        @plsc.parallel_loop(0, 8, unroll=8)
        def _(i, t=t, buf=buf, bas=bas):
            idx2[t, pl.ds(i * 16, 16)] = buf[pl.ds(bas + i * 16, 16)]
    gbufs = [gbuf0, gbuf1]
    gsems = [g0, g1]
    wsems = [w0, w1]
    gd = [None] * 2
    wd = [None] * 2
    for t in range(min(2, ntask)):
        b = t % 2
        gd[b] = pltpu.async_copy(posc.at[idx2.at[t]], gbufs[b], gsems[b])
    for t in range(ntask):
        b = t % 2
        gd[b].wait()
        out, off = tasks[t][2], tasks[t][3]
        wd[b] = pltpu.async_copy(gbufs[b], out.at[pl.ds(off, 128)], wsems[b])
        nt = t + 2
        if nt < ntask:
            wd[b].wait()
            wd[b] = None
            gd[b] = pltpu.async_copy(posc.at[idx2.at[nt]], gbufs[b], gsems[b])
    for b in range(2):
        if wd[b] is not None:
            wd[b].wait()

    pltpu.sync_copy(zf.at[pl.ds(0, 64)], shared_l.at[pl.ds(sid * 64, 64)])
    pltpu.sync_copy(zf.at[pl.ds(64, 64)], shared_g.at[pl.ds(sid * 64, 64)])
    plsc.subcore_barrier()

    ctasks = []
    for j in range(lpt // 128):
        ctasks.append((dstl, j * 128, shared_l))
    for j in range(gpt // 128):
        ctasks.append((dstg, j * 128, shared_g))
    idxCs = [idxC0, idxC1]
    scs = [sc0, sc1]
    sd = [None, None]
    for t, (buf, bas, shared) in enumerate(ctasks):
        b = t % 2
        if sd[b] is not None:
            sd[b].wait()
        _fill_idx(idxCs[b], buf, bas, 8)
        sd[b] = pltpu.async_copy(ones_b, shared.at[idxCs[b]], scs[b], add=True)
    for b in range(2):
        if sd[b] is not None:
            sd[b].wait()
    plsc.subcore_barrier()

    pltpu.sync_copy(shared_l.at[pl.ds(sid * 64, 64)], cntl_part.at[cid, pl.ds(sid * 64, 64)])
    pltpu.sync_copy(shared_g.at[pl.ds(sid * 64, 64)], cntg_part.at[cid, pl.ds(sid * 64, 64)])


def _sc1a(eig, eil, posc, zi, zf, onesf):
    f = functools.partial(
        pl.kernel,
        out_type=(
            jax.ShapeDtypeStruct((N * N // 128, 128), jnp.int32),
            jax.ShapeDtypeStruct((ETOT, 128), jnp.float32),
            jax.ShapeDtypeStruct((ETOT, 128), jnp.float32),
            jax.ShapeDtypeStruct((NC, N, 128), jnp.float32),
            jax.ShapeDtypeStruct((NC, N, 128), jnp.float32),
        ),
        mesh=_get_mesh(),
        compiler_params=pltpu.CompilerParams(needs_layout_passes=False, use_tc_tiling_on_sc=False),
        scratch_types=[
            pltpu.VMEM((N * N // NW // 128, 128), jnp.int32),
            pltpu.VMEM((2048,), jnp.int32),
            pltpu.VMEM((2048,), jnp.int32),
            pltpu.VMEM((2048,), jnp.int32),
            pltpu.VMEM((2048,), jnp.int32),
            pltpu.VMEM((EL // NW,), jnp.int32),
            pltpu.VMEM((EL // NW,), jnp.int32),
            pltpu.VMEM((EG // NW,), jnp.int32),
            pltpu.VMEM((EG // NW,), jnp.int32),
        ] + [pltpu.VMEM((128, 128), jnp.float32)] * 2 + [
            pltpu.VMEM((128, 128), jnp.float32),
            pltpu.VMEM((128,), jnp.int32),
            pltpu.VMEM((128,), jnp.int32),
            pltpu.VMEM((24, 128), jnp.int32),
        ] + [pltpu.SemaphoreType.DMA] * 8 + [
            pltpu.VMEM_SHARED((N, 128), jnp.float32),
            pltpu.VMEM_SHARED((N, 128), jnp.float32),
        ],
    )
    return f(_sc1a_body)(eig, eil, posc, zi, zf, onesf)


def _sc1b_body(idmap, eil, e_table, e_l, srcl, dstl, keyrow, keycol,
               idxA, idxB, lidA, lidB, rowsA, rowsB, erowsA, erowsB,
               gA, gB, tA, tB, wA, wB):
    w = _wid()
    lpt = EL // NW
    nj = lpt // 128
    iota16 = lax.iota(jnp.int32, 16)
    pltpu.sync_copy(eil.at[0, pl.ds(w * lpt, lpt)], srcl)
    pltpu.sync_copy(eil.at[1, pl.ds(w * lpt, lpt)], dstl)
    @plsc.parallel_loop(0, lpt // 16, unroll=8)
    def _(i):
        key = srcl[pl.ds(i * 16, 16)] * N + dstl[pl.ds(i * 16, 16)]
        keyrow[pl.ds(i * 16, 16)] = lax.shift_right_logical(key, 7)
        keycol[pl.ds(i * 16, 16)] = key & 127

    idxs = [idxA, idxB]
    lids = [lidA, lidB]
    rows = [rowsA, rowsB]
    erows = [erowsA, erowsB]
    gs = [gA, gB]
    ts = [tA, tB]
    ws = [wA, wB]

    def start_g(j, b):
        _fill_idx(idxs[b], keyrow, j * 128, 8)
        return pltpu.async_copy(idmap.at[idxs[b]], rows[b], gs[b])

    gd = [None, None]
    wd = [None, None]
    gd[0] = start_g(0, 0)
    for j in range(nj):
        b = j % 2
        gd[b].wait()
        if j + 1 < nj:
            gd[(j + 1) % 2] = start_g(j + 1, (j + 1) % 2)
        for t in range(8):
            r16 = t * 16 + iota16
            c16 = keycol[pl.ds(j * 128 + t * 16, 16)]
            lids[b][pl.ds(t * 16, 16)] = plsc.load_gather(rows[b], [r16, c16])
        if wd[b] is not None:
            wd[b].wait()
            wd[b] = None
        pltpu.async_copy(e_table.at[lids[b]], erows[b], ts[b]).wait()
        wd[b] = pltpu.async_copy(erows[b], e_l.at[pl.ds(w * lpt + j * 128, 128)], ws[b])
    for b in range(2):
        if wd[b] is not None:
            wd[b].wait()


def _sc1b(idmap, eil, e_table):
    f = functools.partial(
        pl.kernel,
        out_type=jax.ShapeDtypeStruct((EL, 128), jnp.float32),
        mesh=_get_mesh(),
        compiler_params=pltpu.CompilerParams(needs_layout_passes=False, use_tc_tiling_on_sc=False),
        scratch_types=[
            pltpu.VMEM((EL // NW,), jnp.int32),
            pltpu.VMEM((EL // NW,), jnp.int32),
            pltpu.VMEM((EL // NW,), jnp.int32),
            pltpu.VMEM((EL // NW,), jnp.int32),
            pltpu.VMEM((128,), jnp.int32),
            pltpu.VMEM((128,), jnp.int32),
            pltpu.VMEM((128,), jnp.int32),
            pltpu.VMEM((128,), jnp.int32),
            pltpu.VMEM((128, 128), jnp.int32),
            pltpu.VMEM((128, 128), jnp.int32),
            pltpu.VMEM((128, 128), jnp.float32),
            pltpu.VMEM((128, 128), jnp.float32),
        ] + [pltpu.SemaphoreType.DMA] * 6,
    )
    return f(_sc1b_body)(idmap, eil, e_table)


def _make_sc2(E):
    ept = E // NW
    nj = ept // 128
    ntask = 2 * nj

    def body(ei, PQ, PS, QD, srcb, dstb, idx2,
             buf0, buf1, buf2, buf3, g0, g1, g2, g3, w0, w1, w2, w3):
        w = _wid()
        bufs = [buf0, buf1, buf2, buf3]
        gsems = [g0, g1, g2, g3]
        wsems = [w0, w1, w2, w3]
        pltpu.sync_copy(ei.at[0, pl.ds(w * ept, ept)], srcb)
        pltpu.sync_copy(ei.at[1, pl.ds(w * ept, ept)], dstb)
        for j in range(nj):
            @plsc.parallel_loop(0, 8, unroll=8)
            def _(i, j=j):
                idx2[j, pl.ds(i * 16, 16)] = srcb[pl.ds(j * 128 + i * 16, 16)]
                idx2[nj + j, pl.ds(i * 16, 16)] = dstb[pl.ds(j * 128 + i * 16, 16)]

        def out_at(t):
            if t < nj:
                return PS.at[pl.ds(w * ept + t * 128, 128)]
            return QD.at[pl.ds(w * ept + (t - nj) * 128, 128)]

        gd = [None] * 4
        wd = [None] * 4
        for t in range(min(4, ntask)):
            b = t % 4
            gd[b] = pltpu.async_copy(PQ.at[idx2.at[t]], bufs[b], gsems[b])
        for t in range(ntask):
            b = t % 4
            gd[b].wait()
            wd[b] = pltpu.async_copy(bufs[b], out_at(t), wsems[b])
            nt = t + 4
            if nt < ntask:
                wd[b].wait()
                wd[b] = None
                gd[b] = pltpu.async_copy(PQ.at[idx2.at[nt]], bufs[b], gsems[b])
        for b in range(4):
            if wd[b] is not None:
                wd[b].wait()

    f = functools.partial(
        pl.kernel,
        out_type=(
            jax.ShapeDtypeStruct((E, 128), jnp.float32),
            jax.ShapeDtypeStruct((E, 128), jnp.float32),
        ),
        mesh=_get_mesh(),
        compiler_params=pltpu.CompilerParams(needs_layout_passes=False, use_tc_tiling_on_sc=False),
        scratch_types=[
            pltpu.VMEM((ept,), jnp.int32),
            pltpu.VMEM((ept,), jnp.int32),
            pltpu.VMEM((ntask, 128), jnp.int32),
        ] + [pltpu.VMEM((128, 128), jnp.float32)] * 4
          + [pltpu.SemaphoreType.DMA] * 8,
    )
    return f(body)


MSGW = SDIM + 3 * VDIM


def _make_sc3(E, W, base):
    ept = E // NW
    ntask = ept // 64

    def body(msg, ei, zrows, agg, dstb,
             idx0, idx1, idx2_, idx3, buf0, buf1, buf2, buf3,
             g0, g1, g2, g3, s0, s1, s2, s3, shared):
        w = _wid()
        cid = lax.axis_index("c")
        sid = lax.axis_index("s")
        bufs = [buf0, buf1, buf2, buf3]
        idxs = [idx0, idx1, idx2_, idx3]
        gsems = [g0, g1, g2, g3]
        ssems = [s0, s1, s2, s3]
        pltpu.sync_copy(zrows, shared.at[pl.ds(sid * 64, 64)])
        plsc.subcore_barrier()
        pltpu.sync_copy(ei.at[1, pl.ds(w * ept, ept)], dstb)

        gd = [None] * 4
        sd = [None] * 4
        for t in range(min(4, ntask)):
            b = t % 4
            gd[b] = pltpu.async_copy(msg.at[pl.ds(base + w * ept + t * 64, 64)],
                                     bufs[b], gsems[b])
        for t in range(ntask):
            b = t % 4
            gd[b].wait()
            _fill_idx(idxs[b], dstb, t * 64, 4)
            sd[b] = pltpu.async_copy(bufs[b], shared.at[idxs[b]], ssems[b], add=True)
            nt = t + 4
            if nt < ntask:
                sd[b].wait()
                sd[b] = None
                gd[b] = pltpu.async_copy(msg.at[pl.ds(base + w * ept + nt * 64, 64)],
                                         bufs[b], gsems[b])
        for b in range(4):
            if sd[b] is not None:
                sd[b].wait()
        plsc.subcore_barrier()
        pltpu.sync_copy(shared.at[pl.ds(sid * 64, 64)], agg.at[cid, pl.ds(sid * 64, 64)])

    f = functools.partial(
        pl.kernel,
        out_type=jax.ShapeDtypeStruct((NC, N, W), jnp.float32),
        mesh=_get_mesh(),
        compiler_params=pltpu.CompilerParams(needs_layout_passes=False, use_tc_tiling_on_sc=False),
        scratch_types=[
            pltpu.VMEM((ept,), jnp.int32),
        ] + [pltpu.VMEM((64,), jnp.int32)] * 4
          + [pltpu.VMEM((64, W), jnp.float32)] * 4
          + [pltpu.SemaphoreType.DMA] * 8
          + [pltpu.VMEM_SHARED((N, W), jnp.float32)],
    )
    return f(body)


def _silu(x):
    return x * (1.0 / (1.0 + jnp.exp(-x)))


def _dot(a, b):
    return jax.lax.dot_general(a, b, (((1,), (0,)), ((), ())),
                               preferred_element_type=jnp.float32)


def _dotT(a, b):
    return jax.lax.dot_general(a, b, (((0,), (0,)), ((), ())),
                               preferred_element_type=jnp.float32)


def _tc_prep_a_body(x, W_atom, b_atom, posp, batch_row, W1s0, W1d0,
                    s0_o, PQ0_o, posc_o):
    s0 = _dot(x[...], W_atom[...]) + b_atom[...]
    s0_o[...] = s0
    PQ0_o[...] = jnp.concatenate([_dot(s0, W1s0[...]), _dot(s0, W1d0[...])], axis=1)
    M = (batch_row[...] == lax.broadcasted_iota(jnp.int32, (NB, N), 0)).astype(jnp.float32)
    cnt_b = jnp.sum(M, axis=1, keepdims=True)
    pos_mean = _dot(M, posp[...]) / jnp.maximum(cnt_b, 1.0)
    posc = posp[...] - _dotT(M, pos_mean)
    posc_o[...] = jnp.concatenate(
        [posc, jnp.zeros((N, 112), jnp.float32)], axis=1)


def _tc_prep_a(x, W_atom, b_atom, posp, batch_row, W1s0, W1d0):
    return pl.pallas_call(
        _tc_prep_a_body,
        out_shape=(
            jax.ShapeDtypeStruct((N, SDIM), jnp.float32),
            jax.ShapeDtypeStruct((N, 128), jnp.float32),
            jax.ShapeDtypeStruct((N, 128), jnp.float32),
        ),
    )(x, W_atom, b_atom, posp, batch_row, W1s0, W1d0)


def _tc_prep_b_body(ea, W_bond, b_bond, out):
    i = pl.program_id(0)
    et = _dot(ea[...], W_bond[...]) + b_bond[...]
    et = jnp.where(i == 0, jnp.zeros_like(et), et)
    out[...] = jnp.concatenate([et, jnp.zeros((et.shape[0], 128 - EDIM), jnp.float32)], axis=1)


def _tc_prep_b(ea8, W_bond8, b_bond):
    nb = NTBL // 512
    return pl.pallas_call(
        _tc_prep_b_body,
        grid=(nb,),
        in_specs=[
            pl.BlockSpec((512, 8), lambda i: (jnp.maximum(i - 1, 0), 0)),
            pl.BlockSpec((8, EDIM), lambda i: (0, 0)),
            pl.BlockSpec((1, EDIM), lambda i: (0, 0)),
        ],
        out_specs=pl.BlockSpec((512, 128), lambda i: (i, 0)),
        out_shape=jax.ShapeDtypeStruct((NTBL, 128), jnp.float32),
    )(ea8, W_bond8, b_bond)


def _tc_geom_body(ps, pd, e, G_o, rn_o):
    psv = ps[...]
    pdv = pd[...]
    r = pdv - psv
    d2 = jnp.sum(r * r, axis=1, keepdims=True)
    a = jnp.sum(psv * pdv, axis=1, keepdims=True)
    d = jnp.sqrt(jnp.maximum(d2, 1e-6))
    rn_o[...] = r / d
    mus = (CUTOFF / (RBF - 1)) * lax.broadcasted_iota(jnp.int32, (1, RBF), 1).astype(jnp.float32)
    rb = jnp.exp(-GAMMA * (d - mus) ** 2)
    G_o[...] = jnp.concatenate(
        [rb, e[:, :EDIM], a, jnp.zeros((rb.shape[0], 128 - RBF - EDIM - 1), jnp.float32)], axis=1)


def _tc_geom(pos_src, pos_dst, e_cat):
    nb = ETOT // 512
    return pl.pallas_call(
        _tc_geom_body,
        grid=(nb,),
        in_specs=[
            pl.BlockSpec((512, 128), lambda i: (i, 0)),
            pl.BlockSpec((512, 128), lambda i: (i, 0)),
            pl.BlockSpec((512, 128), lambda i: (i, 0)),
        ],
        out_specs=(
            pl.BlockSpec((512, 128), lambda i: (i, 0)),
            pl.BlockSpec((512, 128), lambda i: (i, 0)),
        ),
        out_shape=(
            jax.ShapeDtypeStruct((ETOT, 128), jnp.float32),
            jax.ShapeDtypeStruct((ETOT, 128), jnp.float32),
        ),
    )(pos_src, pos_dst, e_cat)


def _tc_et_body(G, W1g, b1, out):
    g = G[...]
    for l in range(L):
        out[l, :, :] = _dot(g, W1g[l]) + b1[l][None, :]


def _tc_et(G, W1g_pad, b1):
    nb = ETOT // 512
    return pl.pallas_call(
        _tc_et_body,
        grid=(nb,),
        in_specs=[
            pl.BlockSpec((512, 128), lambda i: (i, 0)),
            pl.BlockSpec((L, 128, MH), lambda i: (0, 0, 0)),
            pl.BlockSpec((L, MH), lambda i: (0, 0)),
        ],
        out_specs=pl.BlockSpec((L, 512, MH), lambda i: (0, i, 0)),
        out_shape=jax.ShapeDtypeStruct((L, ETOT, MH), jnp.float32),
    )(G, W1g_pad, b1)


def _make_tc_b(E, base, l):
    nb = E // 512

    def body(ET, PS, QD, rn, out):
        u = _silu(ET[0] + PS[:, :MH] + QD[:, MH:])
        rnv = rn[...]
        out[...] = jnp.concatenate(
            [u, u * rnv[:, 0:1], u * rnv[:, 1:2], u * rnv[:, 2:3]], axis=1)

    return pl.pallas_call(
        body,
        grid=(nb,),
        in_specs=[
            pl.BlockSpec((1, 512, MH), lambda i: (l, base // 512 + i, 0)),
            pl.BlockSpec((512, 128), lambda i: (i, 0)),
            pl.BlockSpec((512, 128), lambda i: (i, 0)),
            pl.BlockSpec((512, 128), lambda i: (base // 512 + i, 0)),
        ],
        out_specs=pl.BlockSpec((512, 4 * MH), lambda i: (i, 0)),
        out_shape=jax.ShapeDtypeStruct((E, 4 * MH), jnp.float32),
    )


def _tc_d1_body(s, v, aggL, cntl, Rl, W2s, W2v, b2s, b2v, W1s, W1d,
                s_mid_o, v_mid_o, PQ_o):
    U = aggL[0] + aggL[1]
    cnt = jnp.maximum(cntl[0, :, 0:1] + cntl[1, :, 0:1], 1.0)
    s_mid = s[...] + _dot(U[:, :MH], W2s[...]) + cnt * b2s[...]
    vparts = []
    for k in range(3):
        Rk = Rl[0, :, k:k + 1] + Rl[1, :, k:k + 1]
        vparts.append((_dot(U[:, MH * (k + 1):MH * (k + 2)], W2v[...]) + Rk * b2v[...]) / cnt)
    v_mid_o[...] = v[...] + jnp.concatenate(vparts, axis=1)
    s_mid_o[...] = s_mid
    PQ_o[...] = jnp.concatenate([_dot(s_mid, W1s[...]), _dot(s_mid, W1d[...])], axis=1)


def _tc_d1(s, v, aggL, cntl, Rl, W2s, W2v, b2s, b2v, W1s, W1d):
    return pl.pallas_call(
        _tc_d1_body,
        out_shape=(
            jax.ShapeDtypeStruct((N, SDIM), jnp.float32),
            jax.ShapeDtypeStruct((N, 3 * VDIM), jnp.float32),
            jax.ShapeDtypeStruct((N, 128), jnp.float32),
        ),
    )(s, v, aggL, cntl, Rl, W2s, W2v, b2s, b2v, W1s, W1d)


def _tc_d2_body(s_mid, v_mid, aggG, cntg, Rg, W2s, W2v, b2s, b2v, Wvl, W1s, W1d,
                s_o, v_o, PQ_o):
    U = aggG[0] + aggG[1]
    cnt = jnp.maximum(cntg[0, :, 0:1] + cntg[1, :, 0:1], 1.0)
    s2 = s_mid[...] + _dot(U[:, :MH], W2s[...]) + cnt * b2s[...]
    vparts = []
    for k in range(3):
        Rk = Rg[0, :, k:k + 1] + Rg[1, :, k:k + 1]
        vparts.append((_dot(U[:, MH * (k + 1):MH * (k + 2)], W2v[...]) + Rk * b2v[...]) / cnt)
    v_new = v_mid[...] + jnp.concatenate(vparts, axis=1)
    v_o[...] = v_new
    vn = jnp.sqrt(v_new[:, :VDIM] ** 2 + v_new[:, VDIM:2 * VDIM] ** 2
                  + v_new[:, 2 * VDIM:] ** 2 + 1e-6)
    sp = s2 + _dot(vn, Wvl[...])
    m = jnp.mean(sp, axis=1, keepdims=True)
    c = sp - m
    var = jnp.mean(c * c, axis=1, keepdims=True)
    s_new = c / jnp.sqrt(var + 1e-5)
    s_o[...] = s_new
    PQ_o[...] = jnp.concatenate([_dot(s_new, W1s[...]), _dot(s_new, W1d[...])], axis=1)


def _tc_d2(s_mid, v_mid, aggG, cntg, Rg, W2s, W2v, b2s, b2v, Wvl, W1s, W1d):
    return pl.pallas_call(
        _tc_d2_body,
        out_shape=(
            jax.ShapeDtypeStruct((N, SDIM), jnp.float32),
            jax.ShapeDtypeStruct((N, 3 * VDIM), jnp.float32),
            jax.ShapeDtypeStruct((N, 128), jnp.float32),
        ),
    )(s_mid, v_mid, aggG, cntg, Rg, W2s, W2v, b2s, b2v, Wvl, W1s, W1d)


def _tc_readout_body(s, batch_col, W_lat, b_lat, Wn1, bn1, Wn2, bn2,
                     Wg1, bg1, Wg2, bg2, pooled_o):
    out = _dot(s[...], W_lat[...]) + b_lat[...]
    g1 = _silu(_dot(out, Wg1[...]) + bg1[...])
    gate = _dot(g1, Wg2[...]) + bg2[...]
    nd = _silu(_dot(out, Wn1[...]) + bn1[...])
    nd = _dot(nd, Wn2[...]) + bn2[...]
    MT = (batch_col[:, 0:1] == lax.broadcasted_iota(jnp.int32, (N, NB), 1))
    MTf = MT.astype(jnp.float32)
    masked = jnp.where(MT, jnp.broadcast_to(gate, (N, NB)), -1e30)
    gmax = jnp.max(masked, axis=0, keepdims=True)
    gmax_pn = jax.lax.dot_general(MTf, gmax, (((1,), (1,)), ((), ())),
                                  preferred_element_type=jnp.float32)
    ge = jnp.exp(gate - gmax_pn)
    gden = _dotT(MTf, ge)
    gden_pn = _dot(MTf, gden)
    gate_n = ge / jnp.maximum(gden_pn, 1e-16)
    pooled_o[...] = _dotT(MTf, gate_n * nd)


def _tc_readout(s, batch_col, W_lat, b_lat, Wn1, bn1, Wn2, bn2, Wg1, bg1, Wg2, bg2):
    return pl.pallas_call(
        _tc_readout_body,
        out_shape=jax.ShapeDtypeStruct((NB, LAT), jnp.float32),
    )(s, batch_col, W_lat, b_lat, Wn1, bn1, Wn2, bn2, Wg1, bg1, Wg2, bg2)


@jax.jit
def kernel(x, pos, edge_index_local, edge_index_global, edge_attr_global, batch,
           W_atom, b_atom, W_bond, b_bond, W1, b1, W2, b2, Wv, W_lat, b_lat,
           Wn1, bn1, Wn2, bn2, Wg1, bg1, Wg2, bg2):
    eil = edge_index_local.astype(jnp.int32)
    eig = edge_index_global.astype(jnp.int32)
    batch_i = batch.astype(jnp.int32)
    posp = jnp.pad(pos, ((0, 0), (0, 16 - 3)))
    batch_row = batch_i.reshape(1, N)
    batch_col = jnp.broadcast_to(batch_i.reshape(N, 1), (N, 8))
    ea8 = jnp.pad(edge_attr_global, ((0, 0), (0, 8 - FB)))
    W_bond8 = jnp.pad(W_bond, ((0, 8 - FB), (0, 0)))

    W1s = W1[:, :SDIM, :]
    W1d = W1[:, SDIM:2 * SDIM, :]
    W1g_pad = jnp.zeros((L, 128, MH), jnp.float32)
    W1g_pad = W1g_pad.at[:, :RBF, :].set(W1[:, 2 * SDIM:2 * SDIM + RBF, :])
    W1g_pad = W1g_pad.at[:, RBF:RBF + EDIM, :].set(W1[:, 2 * SDIM + RBF:2 * SDIM + RBF + EDIM, :])
    W1g_pad = W1g_pad.at[:, RBF + EDIM, :].set(W1[:, 2 * SDIM + RBF + EDIM, :])

    s0, PQ, posc = _tc_prep_a(x, W_atom, b_atom.reshape(1, SDIM), posp,
                                batch_row, W1s[0], W1d[0])
    e_table = _tc_prep_b(ea8, W_bond8, b_bond.reshape(1, EDIM))

    zi = jnp.zeros((N * N // NW // 128, 128), jnp.int32)
    zf = jnp.zeros((128, 128), jnp.float32)
    onesf = jnp.ones((128, 128), jnp.float32)
    idmap, pos_src, pos_dst, cntl_part, cntg_part = _sc1a(eig, eil, posc, zi, zf, onesf)
    e_l = _sc1b(idmap, eil, e_table)

    e_cat = jnp.concatenate([e_l, e_table[TBL_OFF:]], axis=0)
    G, rn128 = _tc_geom(pos_src, pos_dst, e_cat)
    ET_all = _tc_et(G, W1g_pad, b1)

    sc2_l = _make_sc2(EL)
    sc2_g = _make_sc2(EG)
    sc3_l = _make_sc3(EL, 4 * MH, 0)
    sc3_g = _make_sc3(EG, 4 * MH, 0)
    z256 = jnp.zeros((64, 4 * MH), jnp.float32)
    z128 = jnp.zeros((64, 128), jnp.float32)
    Rl_part = _make_sc3(EL, 128, 0)(rn128, eil, z128)
    Rg_part = _make_sc3(EG, 128, EL)(rn128, eig, z128)

    W2s = W2[:, :, :SDIM]
    W2v = W2[:, :, SDIM:]
    b2s = b2[:, :SDIM]
    b2v = b2[:, SDIM:]

    s = s0
    v = jnp.zeros((N, 3 * VDIM), jnp.float32)
    for l in range(L):
        PS, QD = sc2_l(eil, PQ)
        msg = _make_tc_b(EL, 0, l)(ET_all, PS, QD, rn128)
        aggL = sc3_l(msg, eil, z256)
        s, v, PQ = _tc_d1(s, v, aggL, cntl_part, Rl_part, W2s[l], W2v[l],
                          b2s[l].reshape(1, SDIM), b2v[l].reshape(1, VDIM),
                          W1s[l], W1d[l])
        PS, QD = sc2_g(eig, PQ)
        msg = _make_tc_b(EG, EL, l)(ET_all, PS, QD, rn128)
        aggG = sc3_g(msg, eig, z256)
        ln = min(l + 1, L - 1)
        s, v, PQ = _tc_d2(s, v, aggG, cntg_part, Rg_part, W2s[l], W2v[l],
                          b2s[l].reshape(1, SDIM), b2v[l].reshape(1, VDIM),
                          Wv[l], W1s[ln], W1d[ln])

    return _tc_readout(s, batch_col, W_lat, b_lat.reshape(1, LAT),
                       Wn1, bn1.reshape(1, LAT), Wn2, bn2.reshape(1, LAT),
                       Wg1, bg1.reshape(1, LAT), Wg2, bg2.reshape(1, 1))
